# baseline (speedup 1.0000x reference)
import functools

import numpy as np
import jax
import jax.numpy as jnp
from jax import lax
from jax.experimental import pallas as pl
from jax.experimental.pallas import tpu as pltpu
from jax.experimental.pallas import tpu_sc as plsc

N = 10000
E = 320000
D = 128
NG = 256
NTYPES = 100
TSTD = 0.0271
TMEAN = 0.6226

NT = 32
NCHUNK = 80
CK = 128
NCS = 160
CKS = 64
NSLOT = 5
NEB = 10
EPAD = NT * NCHUNK * CK
NPAD = 10240
RPT = NPAD // 16
ZR = 128

NBLK = 2000
WBLK = 4096

_mesh = plsc.VectorSubcoreMesh(core_axis_name="c", subcore_axis_name="s")
_f32 = jnp.float32


def _sc_edge_pos_body(pos16, srcg, dstg, d2g, src_v, dst_v, a_v, b_v):
    cid = lax.axis_index("c")
    sid = lax.axis_index("s")
    wid = cid * 16 + sid
    pltpu.sync_copy(srcg.at[wid], src_v)
    pltpu.sync_copy(dstg.at[wid], dst_v)

    @pl.loop(0, NCHUNK)
    def _chunk(c):
        pltpu.sync_copy(pos16.at[src_v.at[c]], a_v)
        pltpu.sync_copy(pos16.at[dst_v.at[c]], b_v)

        @pl.loop(0, CK)
        def _edge(k):
            dvec = a_v[k, :] - b_v[k, :]
            a_v[k, :] = dvec * dvec

        pltpu.sync_copy(a_v, d2g.at[wid, c])


_sc_edge_pos = functools.partial(
    pl.kernel,
    out_type=jax.ShapeDtypeStruct((NT, NCHUNK, CK, 16), _f32),
    mesh=_mesh,
    compiler_params=pltpu.CompilerParams(use_tc_tiling_on_sc=False),
    scratch_types=[
        pltpu.VMEM((NCHUNK, CK), jnp.int32),
        pltpu.VMEM((NCHUNK, CK), jnp.int32),
        pltpu.VMEM((CK, 16), _f32),
        pltpu.VMEM((CK, 16), _f32),
    ],
)(_sc_edge_pos_body)


def _sc_scatter_body(m_hbm, sd_hbm, wg_hbm, agg_hbm,
                     agg_sh, w_v, rows, ebuf, gsem, ssem, esem):
    cid = lax.axis_index("c")
    sid = lax.axis_index("s")
    wid = cid * 16 + sid

    @pl.loop(0, CKS)
    def _zrow(r):
        for j in range(8):
            rows[0, r, pl.ds(j * 16, 16)] = jnp.zeros((16,), _f32)

    for t in range(RPT // CKS):
        pltpu.sync_copy(rows.at[0],
                        agg_sh.at[pl.ds(sid * RPT + t * CKS, CKS)])
    plsc.subcore_barrier()

    pltpu.sync_copy(wg_hbm.at[wid, 0], w_v)
    for e in range(5):
        pltpu.async_copy(sd_hbm.at[wid, e], ebuf.at[e], esem.at[e])
    for s in range(3):
        pltpu.make_async_copy(sd_hbm.at[wid, s], ebuf.at[s],
                              esem.at[s]).wait()
        pltpu.async_copy(m_hbm.at[ebuf.at[s, 0]], rows.at[s], gsem.at[s])

    @pl.loop(0, NCS, step=NEB)
    def _grp(c0):
        for off in range(NEB):
            cc = c0 + off
            s = off % NSLOT
            e = off
            s3 = (off + 3) % NSLOT
            e3 = (off + 3) % NEB
            e5 = (off + 5) % NEB
            sm2 = (off - 2) % NSLOT
            em2 = (off - 2) % NEB

            pltpu.make_async_copy(m_hbm.at[ebuf.at[e, 0]], rows.at[s],
                                  gsem.at[s]).wait()

            @pl.when(cc + 5 < NCS)
            def _pf():
                pltpu.async_copy(sd_hbm.at[wid, cc + 5], ebuf.at[e5],
                                 esem.at[e5])

            @pl.when(cc + 3 < NCS)
            def _gnext():
                @pl.when(cc >= 2)
                def _wsc():
                    pltpu.make_async_copy(
                        rows.at[sm2], agg_sh.at[ebuf.at[em2, 1]],
                        ssem.at[sm2]).wait()
                pltpu.make_async_copy(sd_hbm.at[wid, cc + 3], ebuf.at[e3],
                                      esem.at[e3]).wait()
                pltpu.async_copy(m_hbm.at[ebuf.at[e3, 0]], rows.at[s3],
                                 gsem.at[s3])

            @pl.when(cc == NCS // 2)
            def _rld():
                pltpu.sync_copy(wg_hbm.at[wid, 1], w_v)

            cch = cc % (NCS // 2)

            @pl.loop(0, CKS, step=16)
            def _mul(k0):
                wv = w_v[cch // 2, pl.ds((cch % 2) * CKS + k0, 16)]
                for kk in range(16):
                    wk = wv[kk]
                    for j in range(8):
                        sl = (s, k0 + kk, pl.ds(j * 16, 16))
                        rows[sl] = rows[sl] * wk

            pltpu.async_copy(rows.at[s], agg_sh.at[ebuf.at[e, 1]],
                             ssem.at[s], add=True)

    for cc in range(NCS - 5, NCS):
        s = cc % NSLOT
        e = cc % NEB
        pltpu.make_async_copy(rows.at[s], agg_sh.at[ebuf.at[e, 1]],
                              ssem.at[s]).wait()

    plsc.subcore_barrier()
    for t in range(RPT // CKS):
        pltpu.sync_copy(agg_sh.at[pl.ds(sid * RPT + t * CKS, CKS)],
                        rows.at[0])
        pltpu.sync_copy(rows.at[0],
                        agg_hbm.at[cid, pl.ds(sid * RPT + t * CKS, CKS)])


_sc_scatter = functools.partial(
    pl.kernel,
    out_type=jax.ShapeDtypeStruct((2, NPAD, D), _f32),
    mesh=_mesh,
    scratch_types=[
        pltpu.VMEM_SHARED((NPAD, D), _f32),
        pltpu.VMEM((NCS * CKS // 256, 128), _f32),
        pltpu.VMEM((NSLOT, CKS, D), _f32),
        pltpu.VMEM((NEB, 2, CKS), jnp.int32),
        pltpu.SemaphoreType.DMA((NSLOT,)),
        pltpu.SemaphoreType.DMA((NSLOT,)),
        pltpu.SemaphoreType.DMA((NEB,)),
    ],
)(_sc_scatter_body)


def _tc_embed_body(z_ref, emb_ref, wm_ref, bm_ref, vdw_ref,
                   h_ref, m_ref, vol_ref):
    z = z_ref[...]
    oh = (z == lax.broadcasted_iota(jnp.int32, (NBLK, NTYPES), 1)
          ).astype(_f32)
    h = jnp.dot(oh, emb_ref[...], preferred_element_type=_f32)
    h_ref[...] = h
    m_ref[...] = jnp.maximum(
        jnp.dot(h, wm_ref[...], preferred_element_type=_f32) + bm_ref[...],
        0.0)
    r = vdw_ref[...]
    vol_ref[...] = jnp.dot(oh, (4.0 / 3.0) * np.pi * r * r * r,
                           preferred_element_type=_f32)


def _tc_embed(z2, atom_embed, W_msg, b_msg2, vdw2):
    return pl.pallas_call(
        _tc_embed_body,
        grid=(N // NBLK,),
        in_specs=[
            pl.BlockSpec((NBLK, 1), lambda i: (i, 0)),
            pl.BlockSpec((NTYPES, D), lambda i: (0, 0)),
            pl.BlockSpec((D, D), lambda i: (0, 0)),
            pl.BlockSpec((1, D), lambda i: (0, 0)),
            pl.BlockSpec((NTYPES, 1), lambda i: (0, 0)),
        ],
        out_specs=[
            pl.BlockSpec((NBLK, D), lambda i: (i, 0)),
            pl.BlockSpec((NBLK, D), lambda i: (i, 0)),
            pl.BlockSpec((NBLK, 1), lambda i: (i, 0)),
        ],
        out_shape=[
            jax.ShapeDtypeStruct((N, D), _f32),
            jax.ShapeDtypeStruct((N, D), _f32),
            jax.ShapeDtypeStruct((N, 1), _f32),
        ],
    )(z2, atom_embed, W_msg, b_msg2, vdw2)


def _tc_w_body(d2_ref, g_ref, w_ref):
    s = jnp.dot(d2_ref[...], g_ref[...], preferred_element_type=_f32)
    w = jnp.exp(-jnp.sqrt(s))
    i = pl.program_id(0)
    row = lax.broadcasted_iota(jnp.int32, s.shape, 0)
    col = lax.broadcasted_iota(jnp.int32, s.shape, 1)
    e = (i * WBLK + row) * 8 + col
    w_ref[...] = jnp.where(e < E, w, 0.0)


def _tc_w(d2m, gmat):
    return pl.pallas_call(
        _tc_w_body,
        grid=(EPAD // 8 // WBLK,),
        in_specs=[
            pl.BlockSpec((WBLK, 128), lambda i: (i, 0)),
            pl.BlockSpec((128, 8), lambda i: (0, 0)),
        ],
        out_specs=pl.BlockSpec((WBLK, 8), lambda i: (i, 0)),
        out_shape=jax.ShapeDtypeStruct((EPAD // 8, 8), _f32),
    )(d2m, gmat)


def _tc_round_body(agg_ref, h_ref, wu_ref, bu_ref, wm_ref, bm_ref,
                   hn_ref, mn_ref):
    a = agg_ref[0] + agg_ref[1]
    hn = jnp.maximum(
        jnp.dot(a, wu_ref[...], preferred_element_type=_f32)
        + bu_ref[...] + h_ref[...], 0.0)
    hn_ref[...] = hn
    if mn_ref is not None:
        mn_ref[...] = jnp.maximum(
            jnp.dot(hn, wm_ref[...], preferred_element_type=_f32)
            + bm_ref[...], 0.0)


def _tc_round(agg, h, W_upd, b_upd2, W_msg, b_msg2, last):
    body = (functools.partial(_tc_round_body, mn_ref=None) if last
            else _tc_round_body)
    out_specs = [pl.BlockSpec((NBLK, D), lambda i: (i, 0))]
    out_shape = [jax.ShapeDtypeStruct((N, D), _f32)]
    if not last:
        out_specs.append(pl.BlockSpec((NBLK, D), lambda i: (i, 0)))
        out_shape.append(jax.ShapeDtypeStruct((N, D), _f32))
    return pl.pallas_call(
        body,
        grid=(N // NBLK,),
        in_specs=[
            pl.BlockSpec((2, NBLK, D), lambda i: (0, i, 0)),
            pl.BlockSpec((NBLK, D), lambda i: (i, 0)),
            pl.BlockSpec((D, D), lambda i: (0, 0)),
            pl.BlockSpec((1, D), lambda i: (0, 0)),
            pl.BlockSpec((D, D), lambda i: (0, 0)),
            pl.BlockSpec((1, D), lambda i: (0, 0)),
        ],
        out_specs=out_specs,
        out_shape=out_shape,
    )(agg, h, W_upd, b_upd2, W_msg, b_msg2)


def _tc_pool_body(h_ref, batch_ref, vol_ref, wo_ref, bo_ref,
                  pred_ref, mvol_ref, g_sc, cnt_sc, vol_sc):
    i = pl.program_id(0)

    @pl.when(i == 0)
    def _init():
        g_sc[...] = jnp.zeros_like(g_sc)
        cnt_sc[...] = jnp.zeros_like(cnt_sc)
        vol_sc[...] = jnp.zeros_like(vol_sc)

    ohT = (lax.broadcasted_iota(jnp.int32, (NG, NBLK), 0) == batch_ref[0]
           ).astype(_f32)
    g_sc[...] += jnp.dot(ohT, h_ref[...], preferred_element_type=_f32)
    cnt_sc[...] += jnp.sum(ohT, axis=1, keepdims=True)
    vol_sc[...] += jnp.dot(ohT, vol_ref[...], preferred_element_type=_f32)

    @pl.when(i == N // NBLK - 1)
    def _fin():
        gm = g_sc[...] / jnp.maximum(cnt_sc[...], 1.0)
        pred = jnp.dot(gm, wo_ref[...], preferred_element_type=_f32) \
            + bo_ref[...]
        pred_ref[...] = pred * TSTD + TMEAN
        mvol_ref[...] = vol_sc[...]


def _tc_pool(h, batch2, vol, W_out, b_out2):
    return pl.pallas_call(
        _tc_pool_body,
        grid=(N // NBLK,),
        in_specs=[
            pl.BlockSpec((NBLK, D), lambda i: (i, 0)),
            pl.BlockSpec((1, 1, NBLK), lambda i: (i, 0, 0)),
            pl.BlockSpec((NBLK, 1), lambda i: (i, 0)),
            pl.BlockSpec((D, 1), lambda i: (0, 0)),
            pl.BlockSpec((1, 1), lambda i: (0, 0)),
        ],
        out_specs=[
            pl.BlockSpec((NG, 1), lambda i: (0, 0)),
            pl.BlockSpec((NG, 1), lambda i: (0, 0)),
        ],
        out_shape=[
            jax.ShapeDtypeStruct((NG, 1), _f32),
            jax.ShapeDtypeStruct((NG, 1), _f32),
        ],
        scratch_shapes=[
            pltpu.VMEM((NG, D), _f32),
            pltpu.VMEM((NG, 1), _f32),
            pltpu.VMEM((NG, 1), _f32),
        ],
    )(h, batch2, vol, W_out, b_out2)


_GMAT = np.kron(np.eye(8, dtype=np.float32), np.ones((16, 1), np.float32))


def kernel(z, pos, edge_index, batch, atom_embed, W_msg, b_msg, W_upd, b_upd,
           W_out, b_out, vdw_radii):
    src = edge_index[0].astype(jnp.int32)
    dst = edge_index[1].astype(jnp.int32)
    padn = EPAD - E
    zpad = jnp.zeros((padn,), jnp.int32)
    srcp = jnp.concatenate([src, zpad])
    dstp = jnp.concatenate([dst, zpad])
    srcg = srcp.reshape(NT, NCHUNK, CK)
    dstg = dstp.reshape(NT, NCHUNK, CK)
    pos16 = jnp.pad(pos.astype(_f32), ((0, 0), (0, 13)))
    z2 = z.astype(jnp.int32).reshape(N, 1)
    batch2 = batch.astype(jnp.int32).reshape(N // NBLK, 1, NBLK)
    b_msg2 = b_msg.reshape(1, D)
    b_upd2 = b_upd.reshape(1, D)
    b_out2 = b_out.reshape(1, 1)
    vdw2 = vdw_radii.reshape(NTYPES, 1)
    gmat = jnp.asarray(_GMAT)

    d2g = _sc_edge_pos(pos16, srcg, dstg)
    h, m, vol = _tc_embed(z2, atom_embed, W_msg, b_msg2, vdw2)
    wflat = _tc_w(d2g.reshape(EPAD // 8, 128), gmat)
    wgs = wflat.reshape(NT, 2, NCS * CKS // 256, 128)
    sdg = jnp.concatenate([srcp.reshape(NT, NCS, 1, CKS),
                           dstp.reshape(NT, NCS, 1, CKS)], axis=2)

    for r in range(3):
        agg = _sc_scatter(m, sdg, wgs)
        outs = _tc_round(agg, h, W_upd, b_upd2, W_msg, b_msg2, last=(r == 2))
        if r < 2:
            h, m = outs
        else:
            (h,) = outs

    pred2, mvol2 = _tc_pool(h, batch2, vol, W_out, b_out2)
    return pred2.reshape(NG), mvol2.reshape(NG)


# speedup vs baseline: 1.3878x; 1.3878x over previous
import functools

import numpy as np
import jax
import jax.numpy as jnp
from jax import lax
from jax.experimental import pallas as pl
from jax.experimental.pallas import tpu as pltpu
from jax.experimental.pallas import tpu_sc as plsc

N = 10000
E = 320000
D = 128
NG = 256
NTYPES = 100
TSTD = 0.0271
TMEAN = 0.6226

NT = 32
NCHUNK = 80
CK = 128
NCS = 160
CKS = 64
NSLOT = 4
NST = 2
NEB = 8
EPAD = NT * NCHUNK * CK
NPAD = 10240
RPT = NPAD // 16
ZR = 128

NBLK = 2000
WBLK = 4096

_mesh = plsc.VectorSubcoreMesh(core_axis_name="c", subcore_axis_name="s")
_f32 = jnp.float32


def _sc_edge_pos_body(pos16, srcg, dstg, d2g, src_v, dst_v, a_v, b_v):
    cid = lax.axis_index("c")
    sid = lax.axis_index("s")
    wid = cid * 16 + sid
    pltpu.sync_copy(srcg.at[wid], src_v)
    pltpu.sync_copy(dstg.at[wid], dst_v)

    @pl.loop(0, NCHUNK)
    def _chunk(c):
        pltpu.sync_copy(pos16.at[src_v.at[c]], a_v)
        pltpu.sync_copy(pos16.at[dst_v.at[c]], b_v)

        @pl.loop(0, CK)
        def _edge(k):
            dvec = a_v[k, :] - b_v[k, :]
            a_v[k, :] = dvec * dvec

        pltpu.sync_copy(a_v, d2g.at[wid, c])


_sc_edge_pos = functools.partial(
    pl.kernel,
    out_type=jax.ShapeDtypeStruct((NT, NCHUNK, CK, 16), _f32),
    mesh=_mesh,
    compiler_params=pltpu.CompilerParams(use_tc_tiling_on_sc=False),
    scratch_types=[
        pltpu.VMEM((NCHUNK, CK), jnp.int32),
        pltpu.VMEM((NCHUNK, CK), jnp.int32),
        pltpu.VMEM((CK, 16), _f32),
        pltpu.VMEM((CK, 16), _f32),
    ],
)(_sc_edge_pos_body)


def _sc_scatter_body(m_hbm, sd_hbm, wg_hbm, agg_hbm,
                     agg_sh, w_v, rows, stage, ebuf, gsem, ssem, esem):
    cid = lax.axis_index("c")
    sid = lax.axis_index("s")
    wid = cid * 16 + sid

    @pl.loop(0, CKS)
    def _zrow(r):
        for j in range(8):
            stage[0, r, pl.ds(j * 16, 16)] = jnp.zeros((16,), _f32)

    for t in range(RPT // CKS):
        pltpu.sync_copy(stage.at[0],
                        agg_sh.at[pl.ds(sid * RPT + t * CKS, CKS)])
    plsc.subcore_barrier()

    pltpu.sync_copy(wg_hbm.at[wid, 0], w_v)
    for e in range(5):
        pltpu.async_copy(sd_hbm.at[wid, e], ebuf.at[e], esem.at[e])
    for s in range(3):
        pltpu.make_async_copy(sd_hbm.at[wid, s], ebuf.at[s],
                              esem.at[s]).wait()
        pltpu.async_copy(m_hbm.at[ebuf.at[s, 0]], rows.at[s], gsem.at[s])

    @pl.loop(0, NCS, step=NEB)
    def _grp(c0):
        for off in range(NEB):
            cc = c0 + off
            s = off % NSLOT
            f = off % NST
            e = off % NEB
            s3 = (off + 3) % NSLOT
            e3 = (off + 3) % NEB
            e5 = (off + 5) % NEB
            em2 = (off - 2) % NEB

            pltpu.make_async_copy(m_hbm.at[ebuf.at[e, 0]], rows.at[s],
                                  gsem.at[s]).wait()

            @pl.when(cc + 5 < NCS)
            def _pf():
                pltpu.async_copy(sd_hbm.at[wid, cc + 5], ebuf.at[e5],
                                 esem.at[e5])

            @pl.when(cc + 3 < NCS)
            def _gnext():
                pltpu.make_async_copy(sd_hbm.at[wid, cc + 3], ebuf.at[e3],
                                      esem.at[e3]).wait()
                pltpu.async_copy(m_hbm.at[ebuf.at[e3, 0]], rows.at[s3],
                                 gsem.at[s3])

            @pl.when(cc >= 2)
            def _wsc():
                pltpu.make_async_copy(stage.at[f],
                                      agg_sh.at[ebuf.at[em2, 1]],
                                      ssem.at[f]).wait()

            @pl.when(cc == NCS // 2)
            def _rld():
                pltpu.sync_copy(wg_hbm.at[wid, 1], w_v)

            cch = cc % (NCS // 2)

            @pl.loop(0, CKS, step=16)
            def _mul(k0):
                wv = w_v[cch // 2, pl.ds((cch % 2) * CKS + k0, 16)]
                for kk in range(16):
                    wk = wv[kk]
                    k = k0 + kk
                    for g in range(4):
                        v = rows[s, k, pl.ds(g * 16, 16)]
                        flo = plsc.bitcast(lax.shift_left(v, 16), _f32)
                        fhi = plsc.bitcast(
                            lax.bitwise_and(v, jnp.int32(-65536)), _f32)
                        stage[f, k, pl.ds(g * 32, 16)] = flo * wk
                        stage[f, k, pl.ds(g * 32 + 16, 16)] = fhi * wk

            pltpu.async_copy(stage.at[f], agg_sh.at[ebuf.at[e, 1]],
                             ssem.at[f], add=True)

    for cc in range(NCS - 2, NCS):
        f = cc % NST
        e = cc % NEB
        pltpu.make_async_copy(stage.at[f], agg_sh.at[ebuf.at[e, 1]],
                              ssem.at[f]).wait()

    plsc.subcore_barrier()
    for t in range(RPT // CKS):
        pltpu.sync_copy(agg_sh.at[pl.ds(sid * RPT + t * CKS, CKS)],
                        stage.at[0])
        pltpu.sync_copy(stage.at[0],
                        agg_hbm.at[cid, pl.ds(sid * RPT + t * CKS, CKS)])


_sc_scatter = functools.partial(
    pl.kernel,
    out_type=jax.ShapeDtypeStruct((2, NPAD, D), _f32),
    mesh=_mesh,
    compiler_params=pltpu.CompilerParams(use_tc_tiling_on_sc=False,
                                         needs_layout_passes=False),
    scratch_types=[
        pltpu.VMEM_SHARED((NPAD, D), _f32),
        pltpu.VMEM((NCS * CKS // 256, 128), _f32),
        pltpu.VMEM((NSLOT, CKS, D // 2), jnp.int32),
        pltpu.VMEM((NST, CKS, D), _f32),
        pltpu.VMEM((NEB, 2, CKS), jnp.int32),
        pltpu.SemaphoreType.DMA((NSLOT,)),
        pltpu.SemaphoreType.DMA((NST,)),
        pltpu.SemaphoreType.DMA((NEB,)),
    ],
)(_sc_scatter_body)


def _tc_embed_body(z_ref, emb_ref, wm_ref, bm_ref, vdw_ref,
                   h_ref, m_ref, vol_ref):
    z = z_ref[...]
    oh = (z == lax.broadcasted_iota(jnp.int32, (NBLK, NTYPES), 1)
          ).astype(_f32)
    h = jnp.dot(oh, emb_ref[...], preferred_element_type=_f32)
    h_ref[...] = h
    m_ref[...] = jnp.maximum(
        jnp.dot(h, wm_ref[...], preferred_element_type=_f32) + bm_ref[...],
        0.0).astype(jnp.bfloat16)
    r = vdw_ref[...]
    vol_ref[...] = jnp.dot(oh, (4.0 / 3.0) * np.pi * r * r * r,
                           preferred_element_type=_f32)


def _tc_embed(z2, atom_embed, W_msg, b_msg2, vdw2):
    return pl.pallas_call(
        _tc_embed_body,
        grid=(N // NBLK,),
        in_specs=[
            pl.BlockSpec((NBLK, 1), lambda i: (i, 0)),
            pl.BlockSpec((NTYPES, D), lambda i: (0, 0)),
            pl.BlockSpec((D, D), lambda i: (0, 0)),
            pl.BlockSpec((1, D), lambda i: (0, 0)),
            pl.BlockSpec((NTYPES, 1), lambda i: (0, 0)),
        ],
        out_specs=[
            pl.BlockSpec((NBLK, D), lambda i: (i, 0)),
            pl.BlockSpec((NBLK, D), lambda i: (i, 0)),
            pl.BlockSpec((NBLK, 1), lambda i: (i, 0)),
        ],
        out_shape=[
            jax.ShapeDtypeStruct((N, D), _f32),
            jax.ShapeDtypeStruct((N, D), jnp.bfloat16),
            jax.ShapeDtypeStruct((N, 1), _f32),
        ],
    )(z2, atom_embed, W_msg, b_msg2, vdw2)


def _tc_w_body(d2_ref, g_ref, w_ref):
    s = jnp.dot(d2_ref[...], g_ref[...], preferred_element_type=_f32)
    w = jnp.exp(-jnp.sqrt(s))
    i = pl.program_id(0)
    row = lax.broadcasted_iota(jnp.int32, s.shape, 0)
    col = lax.broadcasted_iota(jnp.int32, s.shape, 1)
    e = (i * WBLK + row) * 8 + col
    w_ref[...] = jnp.where(e < E, w, 0.0)


def _tc_w(d2m, gmat):
    return pl.pallas_call(
        _tc_w_body,
        grid=(EPAD // 8 // WBLK,),
        in_specs=[
            pl.BlockSpec((WBLK, 128), lambda i: (i, 0)),
            pl.BlockSpec((128, 8), lambda i: (0, 0)),
        ],
        out_specs=pl.BlockSpec((WBLK, 8), lambda i: (i, 0)),
        out_shape=jax.ShapeDtypeStruct((EPAD // 8, 8), _f32),
    )(d2m, gmat)


def _tc_round_body(agg_ref, h_ref, wu_ref, bu_ref, wm_ref, bm_ref,
                   hn_ref, mn_ref):
    a = agg_ref[0] + agg_ref[1]
    hn = jnp.maximum(
        jnp.dot(a, wu_ref[...], preferred_element_type=_f32)
        + bu_ref[...] + h_ref[...], 0.0)
    hn_ref[...] = hn
    if mn_ref is not None:
        mn_ref[...] = jnp.maximum(
            jnp.dot(hn, wm_ref[...], preferred_element_type=_f32)
            + bm_ref[...], 0.0).astype(jnp.bfloat16)


def _tc_round(agg, h, W_upd, b_upd2, W_msg, b_msg2, last):
    body = (functools.partial(_tc_round_body, mn_ref=None) if last
            else _tc_round_body)
    out_specs = [pl.BlockSpec((NBLK, D), lambda i: (i, 0))]
    out_shape = [jax.ShapeDtypeStruct((N, D), _f32)]
    if not last:
        out_specs.append(pl.BlockSpec((NBLK, D), lambda i: (i, 0)))
        out_shape.append(jax.ShapeDtypeStruct((N, D), jnp.bfloat16))
    return pl.pallas_call(
        body,
        grid=(N // NBLK,),
        in_specs=[
            pl.BlockSpec((2, NBLK, D), lambda i: (0, i, 0)),
            pl.BlockSpec((NBLK, D), lambda i: (i, 0)),
            pl.BlockSpec((D, D), lambda i: (0, 0)),
            pl.BlockSpec((1, D), lambda i: (0, 0)),
            pl.BlockSpec((D, D), lambda i: (0, 0)),
            pl.BlockSpec((1, D), lambda i: (0, 0)),
        ],
        out_specs=out_specs,
        out_shape=out_shape,
    )(agg, h, W_upd, b_upd2, W_msg, b_msg2)


def _tc_pool_body(h_ref, batch_ref, vol_ref, wo_ref, bo_ref,
                  pred_ref, mvol_ref, g_sc, cnt_sc, vol_sc):
    i = pl.program_id(0)

    @pl.when(i == 0)
    def _init():
        g_sc[...] = jnp.zeros_like(g_sc)
        cnt_sc[...] = jnp.zeros_like(cnt_sc)
        vol_sc[...] = jnp.zeros_like(vol_sc)

    ohT = (lax.broadcasted_iota(jnp.int32, (NG, NBLK), 0) == batch_ref[0]
           ).astype(_f32)
    g_sc[...] += jnp.dot(ohT, h_ref[...], preferred_element_type=_f32)
    cnt_sc[...] += jnp.sum(ohT, axis=1, keepdims=True)
    vol_sc[...] += jnp.dot(ohT, vol_ref[...], preferred_element_type=_f32)

    @pl.when(i == N // NBLK - 1)
    def _fin():
        gm = g_sc[...] / jnp.maximum(cnt_sc[...], 1.0)
        pred = jnp.dot(gm, wo_ref[...], preferred_element_type=_f32) \
            + bo_ref[...]
        pred_ref[...] = pred * TSTD + TMEAN
        mvol_ref[...] = vol_sc[...]


def _tc_pool(h, batch2, vol, W_out, b_out2):
    return pl.pallas_call(
        _tc_pool_body,
        grid=(N // NBLK,),
        in_specs=[
            pl.BlockSpec((NBLK, D), lambda i: (i, 0)),
            pl.BlockSpec((1, 1, NBLK), lambda i: (i, 0, 0)),
            pl.BlockSpec((NBLK, 1), lambda i: (i, 0)),
            pl.BlockSpec((D, 1), lambda i: (0, 0)),
            pl.BlockSpec((1, 1), lambda i: (0, 0)),
        ],
        out_specs=[
            pl.BlockSpec((NG, 1), lambda i: (0, 0)),
            pl.BlockSpec((NG, 1), lambda i: (0, 0)),
        ],
        out_shape=[
            jax.ShapeDtypeStruct((NG, 1), _f32),
            jax.ShapeDtypeStruct((NG, 1), _f32),
        ],
        scratch_shapes=[
            pltpu.VMEM((NG, D), _f32),
            pltpu.VMEM((NG, 1), _f32),
            pltpu.VMEM((NG, 1), _f32),
        ],
    )(h, batch2, vol, W_out, b_out2)


_GMAT = np.kron(np.eye(8, dtype=np.float32), np.ones((16, 1), np.float32))

_PERM = np.zeros(128, np.int32)
for _g in range(4):
    for _i in range(16):
        _PERM[32 * _g + 2 * _i] = 32 * _g + _i
        _PERM[32 * _g + 2 * _i + 1] = 32 * _g + 16 + _i


def kernel(z, pos, edge_index, batch, atom_embed, W_msg, b_msg, W_upd, b_upd,
           W_out, b_out, vdw_radii):
    src = edge_index[0].astype(jnp.int32)
    dst = edge_index[1].astype(jnp.int32)
    padn = EPAD - E
    zpad = jnp.zeros((padn,), jnp.int32)
    srcp = jnp.concatenate([src, zpad])
    dstp = jnp.concatenate([dst, zpad])
    srcg = srcp.reshape(NT, NCHUNK, CK)
    dstg = dstp.reshape(NT, NCHUNK, CK)
    pos16 = jnp.pad(pos.astype(_f32), ((0, 0), (0, 13)))
    z2 = z.astype(jnp.int32).reshape(N, 1)
    batch2 = batch.astype(jnp.int32).reshape(N // NBLK, 1, NBLK)
    perm = jnp.asarray(_PERM)
    W_msg = W_msg[:, perm]
    b_msg = b_msg[perm]
    b_msg2 = b_msg.reshape(1, D)
    b_upd2 = b_upd.reshape(1, D)
    b_out2 = b_out.reshape(1, 1)
    vdw2 = vdw_radii.reshape(NTYPES, 1)
    gmat = jnp.asarray(_GMAT)

    d2g = _sc_edge_pos(pos16, srcg, dstg)
    h, m, vol = _tc_embed(z2, atom_embed, W_msg, b_msg2, vdw2)
    wflat = _tc_w(d2g.reshape(EPAD // 8, 128), gmat)
    wgs = wflat.reshape(NT, 2, NCS * CKS // 256, 128)
    sdg = jnp.concatenate([srcp.reshape(NT, NCS, 1, CKS),
                           dstp.reshape(NT, NCS, 1, CKS)], axis=2)

    for r in range(3):
        m_i32 = lax.bitcast_convert_type(
            m.reshape(N, D // 2, 2), jnp.int32)
        agg = _sc_scatter(m_i32, sdg, wgs)
        outs = _tc_round(agg, h, W_upd, b_upd2, W_msg, b_msg2, last=(r == 2))
        if r < 2:
            h, m = outs
        else:
            (h,) = outs

    pred2, mvol2 = _tc_pool(h, batch2, vol, W_out, b_out2)
    return pred2.reshape(NG), mvol2.reshape(NG)


# speedup vs baseline: 1.4792x; 1.0659x over previous
import functools

import numpy as np
import jax
import jax.numpy as jnp
from jax import lax
from jax.experimental import pallas as pl
from jax.experimental.pallas import tpu as pltpu
from jax.experimental.pallas import tpu_sc as plsc

N = 10000
E = 320000
D = 128
NG = 256
NTYPES = 100
TSTD = 0.0271
TMEAN = 0.6226

NT = 32
NCHUNK = 80
CK = 128
NCS = 160
CKS = 64
NSLOT = 4
NST = 2
NEB = 8
EPAD = NT * NCHUNK * CK
NPAD = 10240
RPT = NPAD // 16
ZR = 128

NBLK = 2000
WBLK = 4096

_mesh = plsc.VectorSubcoreMesh(core_axis_name="c", subcore_axis_name="s")
_f32 = jnp.float32


def _sc_edge_pos_body(pos16, srcg, dstg, d2g, src_v, dst_v, a_v, b_v):
    cid = lax.axis_index("c")
    sid = lax.axis_index("s")
    wid = cid * 16 + sid
    pltpu.sync_copy(srcg.at[wid], src_v)
    pltpu.sync_copy(dstg.at[wid], dst_v)

    @pl.loop(0, NCHUNK)
    def _chunk(c):
        pltpu.sync_copy(pos16.at[src_v.at[c]], a_v)
        pltpu.sync_copy(pos16.at[dst_v.at[c]], b_v)

        @pl.loop(0, CK)
        def _edge(k):
            dvec = a_v[k, :] - b_v[k, :]
            a_v[k, :] = dvec * dvec

        pltpu.sync_copy(a_v, d2g.at[wid, c])


_sc_edge_pos = functools.partial(
    pl.kernel,
    out_type=jax.ShapeDtypeStruct((NT, NCHUNK, CK, 16), _f32),
    mesh=_mesh,
    compiler_params=pltpu.CompilerParams(use_tc_tiling_on_sc=False),
    scratch_types=[
        pltpu.VMEM((NCHUNK, CK), jnp.int32),
        pltpu.VMEM((NCHUNK, CK), jnp.int32),
        pltpu.VMEM((CK, 16), _f32),
        pltpu.VMEM((CK, 16), _f32),
    ],
)(_sc_edge_pos_body)


def _sc_scatter_body(m_hbm, sd_hbm, wg_hbm, agg_hbm,
                     agg_sh, w_v, rows, stage, ebuf, gsem, ssem, esem):
    cid = lax.axis_index("c")
    sid = lax.axis_index("s")
    wid = cid * 16 + sid

    @pl.loop(0, CKS)
    def _zrow(r):
        for j in range(8):
            stage[0, r, pl.ds(j * 16, 16)] = jnp.zeros((16,), _f32)

    for t in range(RPT // CKS):
        pltpu.sync_copy(stage.at[0],
                        agg_sh.at[pl.ds(sid * RPT + t * CKS, CKS)])
    plsc.subcore_barrier()

    pltpu.sync_copy(wg_hbm.at[wid, 0], w_v)
    for e in range(5):
        pltpu.async_copy(sd_hbm.at[wid, e], ebuf.at[e], esem.at[e])
    for s in range(3):
        pltpu.make_async_copy(sd_hbm.at[wid, s], ebuf.at[s],
                              esem.at[s]).wait()
        pltpu.async_copy(m_hbm.at[ebuf.at[s, 0]], rows.at[s], gsem.at[s])

    @pl.loop(0, NCS, step=NEB)
    def _grp(c0):
        for off in range(NEB):
            cc = c0 + off
            s = off % NSLOT
            f = off % NST
            e = off % NEB
            s3 = (off + 3) % NSLOT
            e3 = (off + 3) % NEB
            e5 = (off + 5) % NEB
            em2 = (off - 2) % NEB

            pltpu.make_async_copy(m_hbm.at[ebuf.at[e, 0]], rows.at[s],
                                  gsem.at[s]).wait()

            @pl.when(cc + 5 < NCS)
            def _pf():
                pltpu.async_copy(sd_hbm.at[wid, cc + 5], ebuf.at[e5],
                                 esem.at[e5])

            @pl.when(cc + 3 < NCS)
            def _gnext():
                pltpu.make_async_copy(sd_hbm.at[wid, cc + 3], ebuf.at[e3],
                                      esem.at[e3]).wait()
                pltpu.async_copy(m_hbm.at[ebuf.at[e3, 0]], rows.at[s3],
                                 gsem.at[s3])

            @pl.when(cc >= 2)
            def _wsc():
                pltpu.make_async_copy(stage.at[f],
                                      agg_sh.at[ebuf.at[em2, 1]],
                                      ssem.at[f]).wait()

            @pl.when(cc == NCS // 2)
            def _rld():
                pltpu.sync_copy(wg_hbm.at[wid, 1], w_v)

            cch = cc % (NCS // 2)

            @pl.loop(0, CKS, step=16)
            def _mul(k0):
                wv = w_v[cch // 2, pl.ds((cch % 2) * CKS + k0, 16)]
                for kk in range(16):
                    wk = wv[kk]
                    k = k0 + kk
                    for g in range(4):
                        v = rows[s, k, pl.ds(g * 16, 16)]
                        flo = plsc.bitcast(lax.shift_left(v, 16), _f32)
                        fhi = plsc.bitcast(v, _f32)
                        stage[f, k, pl.ds(g * 32, 16)] = flo * wk
                        stage[f, k, pl.ds(g * 32 + 16, 16)] = fhi * wk

            pltpu.async_copy(stage.at[f], agg_sh.at[ebuf.at[e, 1]],
                             ssem.at[f], add=True)

    for cc in range(NCS - 2, NCS):
        f = cc % NST
        e = cc % NEB
        pltpu.make_async_copy(stage.at[f], agg_sh.at[ebuf.at[e, 1]],
                              ssem.at[f]).wait()

    plsc.subcore_barrier()
    for t in range(RPT // CKS):
        pltpu.sync_copy(agg_sh.at[pl.ds(sid * RPT + t * CKS, CKS)],
                        stage.at[0])
        pltpu.sync_copy(stage.at[0],
                        agg_hbm.at[cid, pl.ds(sid * RPT + t * CKS, CKS)])


_sc_scatter = functools.partial(
    pl.kernel,
    out_type=jax.ShapeDtypeStruct((2, NPAD, D), _f32),
    mesh=_mesh,
    compiler_params=pltpu.CompilerParams(use_tc_tiling_on_sc=False,
                                         needs_layout_passes=False),
    scratch_types=[
        pltpu.VMEM_SHARED((NPAD, D), _f32),
        pltpu.VMEM((NCS * CKS // 256, 128), _f32),
        pltpu.VMEM((NSLOT, CKS, D // 2), jnp.int32),
        pltpu.VMEM((NST, CKS, D), _f32),
        pltpu.VMEM((NEB, 2, CKS), jnp.int32),
        pltpu.SemaphoreType.DMA((NSLOT,)),
        pltpu.SemaphoreType.DMA((NST,)),
        pltpu.SemaphoreType.DMA((NEB,)),
    ],
)(_sc_scatter_body)


def _tc_embed_body(z_ref, emb_ref, wm_ref, bm_ref, vdw_ref,
                   h_ref, m_ref, vol_ref):
    z = z_ref[...]
    oh = (z == lax.broadcasted_iota(jnp.int32, (NBLK, NTYPES), 1)
          ).astype(_f32)
    h = jnp.dot(oh, emb_ref[...], preferred_element_type=_f32)
    h_ref[...] = h
    m_ref[...] = jnp.maximum(
        jnp.dot(h, wm_ref[...], preferred_element_type=_f32) + bm_ref[...],
        0.0).astype(jnp.bfloat16)
    r = vdw_ref[...]
    vol_ref[...] = jnp.dot(oh, (4.0 / 3.0) * np.pi * r * r * r,
                           preferred_element_type=_f32)


def _tc_embed(z2, atom_embed, W_msg, b_msg2, vdw2):
    return pl.pallas_call(
        _tc_embed_body,
        grid=(N // NBLK,),
        in_specs=[
            pl.BlockSpec((NBLK, 1), lambda i: (i, 0)),
            pl.BlockSpec((NTYPES, D), lambda i: (0, 0)),
            pl.BlockSpec((D, D), lambda i: (0, 0)),
            pl.BlockSpec((1, D), lambda i: (0, 0)),
            pl.BlockSpec((NTYPES, 1), lambda i: (0, 0)),
        ],
        out_specs=[
            pl.BlockSpec((NBLK, D), lambda i: (i, 0)),
            pl.BlockSpec((NBLK, D), lambda i: (i, 0)),
            pl.BlockSpec((NBLK, 1), lambda i: (i, 0)),
        ],
        out_shape=[
            jax.ShapeDtypeStruct((N, D), _f32),
            jax.ShapeDtypeStruct((N, D), jnp.bfloat16),
            jax.ShapeDtypeStruct((N, 1), _f32),
        ],
    )(z2, atom_embed, W_msg, b_msg2, vdw2)


def _tc_w_body(d2_ref, g_ref, w_ref):
    s = jnp.dot(d2_ref[...], g_ref[...], preferred_element_type=_f32)
    w = jnp.exp(-jnp.sqrt(s))
    i = pl.program_id(0)
    row = lax.broadcasted_iota(jnp.int32, s.shape, 0)
    col = lax.broadcasted_iota(jnp.int32, s.shape, 1)
    e = (i * WBLK + row) * 8 + col
    w_ref[...] = jnp.where(e < E, w, 0.0)


def _tc_w(d2m, gmat):
    return pl.pallas_call(
        _tc_w_body,
        grid=(EPAD // 8 // WBLK,),
        in_specs=[
            pl.BlockSpec((WBLK, 128), lambda i: (i, 0)),
            pl.BlockSpec((128, 8), lambda i: (0, 0)),
        ],
        out_specs=pl.BlockSpec((WBLK, 8), lambda i: (i, 0)),
        out_shape=jax.ShapeDtypeStruct((EPAD // 8, 8), _f32),
    )(d2m, gmat)


def _tc_round_body(agg_ref, h_ref, wu_ref, bu_ref, wm_ref, bm_ref,
                   hn_ref, mn_ref):
    a = agg_ref[0] + agg_ref[1]
    hn = jnp.maximum(
        jnp.dot(a, wu_ref[...], preferred_element_type=_f32)
        + bu_ref[...] + h_ref[...], 0.0)
    hn_ref[...] = hn
    if mn_ref is not None:
        mn_ref[...] = jnp.maximum(
            jnp.dot(hn, wm_ref[...], preferred_element_type=_f32)
            + bm_ref[...], 0.0).astype(jnp.bfloat16)


def _tc_round(agg, h, W_upd, b_upd2, W_msg, b_msg2, last):
    body = (functools.partial(_tc_round_body, mn_ref=None) if last
            else _tc_round_body)
    out_specs = [pl.BlockSpec((NBLK, D), lambda i: (i, 0))]
    out_shape = [jax.ShapeDtypeStruct((N, D), _f32)]
    if not last:
        out_specs.append(pl.BlockSpec((NBLK, D), lambda i: (i, 0)))
        out_shape.append(jax.ShapeDtypeStruct((N, D), jnp.bfloat16))
    return pl.pallas_call(
        body,
        grid=(N // NBLK,),
        in_specs=[
            pl.BlockSpec((2, NBLK, D), lambda i: (0, i, 0)),
            pl.BlockSpec((NBLK, D), lambda i: (i, 0)),
            pl.BlockSpec((D, D), lambda i: (0, 0)),
            pl.BlockSpec((1, D), lambda i: (0, 0)),
            pl.BlockSpec((D, D), lambda i: (0, 0)),
            pl.BlockSpec((1, D), lambda i: (0, 0)),
        ],
        out_specs=out_specs,
        out_shape=out_shape,
    )(agg, h, W_upd, b_upd2, W_msg, b_msg2)


def _tc_pool_body(h_ref, batch_ref, vol_ref, wo_ref, bo_ref,
                  pred_ref, mvol_ref, g_sc, cnt_sc, vol_sc):
    i = pl.program_id(0)

    @pl.when(i == 0)
    def _init():
        g_sc[...] = jnp.zeros_like(g_sc)
        cnt_sc[...] = jnp.zeros_like(cnt_sc)
        vol_sc[...] = jnp.zeros_like(vol_sc)

    ohT = (lax.broadcasted_iota(jnp.int32, (NG, NBLK), 0) == batch_ref[0]
           ).astype(_f32)
    g_sc[...] += jnp.dot(ohT, h_ref[...], preferred_element_type=_f32)
    cnt_sc[...] += jnp.sum(ohT, axis=1, keepdims=True)
    vol_sc[...] += jnp.dot(ohT, vol_ref[...], preferred_element_type=_f32)

    @pl.when(i == N // NBLK - 1)
    def _fin():
        gm = g_sc[...] / jnp.maximum(cnt_sc[...], 1.0)
        pred = jnp.dot(gm, wo_ref[...], preferred_element_type=_f32) \
            + bo_ref[...]
        pred_ref[...] = pred * TSTD + TMEAN
        mvol_ref[...] = vol_sc[...]


def _tc_pool(h, batch2, vol, W_out, b_out2):
    return pl.pallas_call(
        _tc_pool_body,
        grid=(N // NBLK,),
        in_specs=[
            pl.BlockSpec((NBLK, D), lambda i: (i, 0)),
            pl.BlockSpec((1, 1, NBLK), lambda i: (i, 0, 0)),
            pl.BlockSpec((NBLK, 1), lambda i: (i, 0)),
            pl.BlockSpec((D, 1), lambda i: (0, 0)),
            pl.BlockSpec((1, 1), lambda i: (0, 0)),
        ],
        out_specs=[
            pl.BlockSpec((NG, 1), lambda i: (0, 0)),
            pl.BlockSpec((NG, 1), lambda i: (0, 0)),
        ],
        out_shape=[
            jax.ShapeDtypeStruct((NG, 1), _f32),
            jax.ShapeDtypeStruct((NG, 1), _f32),
        ],
        scratch_shapes=[
            pltpu.VMEM((NG, D), _f32),
            pltpu.VMEM((NG, 1), _f32),
            pltpu.VMEM((NG, 1), _f32),
        ],
    )(h, batch2, vol, W_out, b_out2)


_GMAT = np.kron(np.eye(8, dtype=np.float32), np.ones((16, 1), np.float32))

_PERM = np.zeros(128, np.int32)
for _g in range(4):
    for _i in range(16):
        _PERM[32 * _g + 2 * _i] = 32 * _g + _i
        _PERM[32 * _g + 2 * _i + 1] = 32 * _g + 16 + _i


def kernel(z, pos, edge_index, batch, atom_embed, W_msg, b_msg, W_upd, b_upd,
           W_out, b_out, vdw_radii):
    src = edge_index[0].astype(jnp.int32)
    dst = edge_index[1].astype(jnp.int32)
    padn = EPAD - E
    zpad = jnp.zeros((padn,), jnp.int32)
    srcp = jnp.concatenate([src, zpad])
    dstp = jnp.concatenate([dst, zpad])
    srcg = srcp.reshape(NT, NCHUNK, CK)
    dstg = dstp.reshape(NT, NCHUNK, CK)
    pos16 = jnp.pad(pos.astype(_f32), ((0, 0), (0, 13)))
    z2 = z.astype(jnp.int32).reshape(N, 1)
    batch2 = batch.astype(jnp.int32).reshape(N // NBLK, 1, NBLK)
    perm = jnp.asarray(_PERM)
    W_msg = W_msg[:, perm]
    b_msg = b_msg[perm]
    b_msg2 = b_msg.reshape(1, D)
    b_upd2 = b_upd.reshape(1, D)
    b_out2 = b_out.reshape(1, 1)
    vdw2 = vdw_radii.reshape(NTYPES, 1)
    gmat = jnp.asarray(_GMAT)

    d2g = _sc_edge_pos(pos16, srcg, dstg)
    h, m, vol = _tc_embed(z2, atom_embed, W_msg, b_msg2, vdw2)
    wflat = _tc_w(d2g.reshape(EPAD // 8, 128), gmat)
    wgs = wflat.reshape(NT, 2, NCS * CKS // 256, 128)
    sdg = jnp.concatenate([srcp.reshape(NT, NCS, 1, CKS),
                           dstp.reshape(NT, NCS, 1, CKS)], axis=2)

    for r in range(3):
        m_i32 = lax.bitcast_convert_type(
            m.reshape(N, D // 2, 2), jnp.int32)
        agg = _sc_scatter(m_i32, sdg, wgs)
        outs = _tc_round(agg, h, W_upd, b_upd2, W_msg, b_msg2, last=(r == 2))
        if r < 2:
            h, m = outs
        else:
            (h,) = outs

    pred2, mvol2 = _tc_pool(h, batch2, vol, W_out, b_out2)
    return pred2.reshape(NG), mvol2.reshape(NG)


# speedup vs baseline: 1.5414x; 1.0420x over previous
import functools

import numpy as np
import jax
import jax.numpy as jnp
from jax import lax
from jax.experimental import pallas as pl
from jax.experimental.pallas import tpu as pltpu
from jax.experimental.pallas import tpu_sc as plsc

N = 10000
E = 320000
D = 128
NG = 256
NTYPES = 100
TSTD = 0.0271
TMEAN = 0.6226

NT = 32
NCHUNK = 80
CK = 128
NCS = 160
CKS = 64
NSLOT = 4
NST = 2
NEB = 8
EPAD = NT * NCHUNK * CK
NPAD = 10240
RPT = NPAD // 16
ZR = 128

NBLK = 2000
WBLK = 4096

_mesh = plsc.VectorSubcoreMesh(core_axis_name="c", subcore_axis_name="s")
_f32 = jnp.float32


def _sc_edge_pos_body(pos16, srcg, dstg, d2g, src_v, dst_v, a_v, b_v):
    cid = lax.axis_index("c")
    sid = lax.axis_index("s")
    wid = cid * 16 + sid
    pltpu.sync_copy(srcg.at[wid], src_v)
    pltpu.sync_copy(dstg.at[wid], dst_v)

    @pl.loop(0, NCHUNK)
    def _chunk(c):
        pltpu.sync_copy(pos16.at[src_v.at[c]], a_v)
        pltpu.sync_copy(pos16.at[dst_v.at[c]], b_v)

        @pl.loop(0, CK)
        def _edge(k):
            dvec = a_v[k, :] - b_v[k, :]
            a_v[k, :] = dvec * dvec

        pltpu.sync_copy(a_v, d2g.at[wid, c])


_sc_edge_pos = functools.partial(
    pl.kernel,
    out_type=jax.ShapeDtypeStruct((NT, NCHUNK, CK, 16), _f32),
    mesh=_mesh,
    compiler_params=pltpu.CompilerParams(use_tc_tiling_on_sc=False),
    scratch_types=[
        pltpu.VMEM((NCHUNK, CK), jnp.int32),
        pltpu.VMEM((NCHUNK, CK), jnp.int32),
        pltpu.VMEM((CK, 16), _f32),
        pltpu.VMEM((CK, 16), _f32),
    ],
)(_sc_edge_pos_body)


def _sc_scatter_body(m_hbm, sd_hbm, wg_hbm, agg_hbm,
                     agg_sh, w_v, rows, stage, ebuf, gsem, ssem, esem):
    cid = lax.axis_index("c")
    sid = lax.axis_index("s")
    wid = cid * 16 + sid

    @pl.loop(0, CKS)
    def _zrow(r):
        for j in range(8):
            stage[0, r, pl.ds(j * 16, 16)] = jnp.zeros((16,), _f32)

    for t in range(RPT // CKS):
        pltpu.sync_copy(stage.at[0],
                        agg_sh.at[pl.ds(sid * RPT + t * CKS, CKS)])
    plsc.subcore_barrier()

    pltpu.sync_copy(wg_hbm.at[wid, 0], w_v)
    for e in range(5):
        pltpu.async_copy(sd_hbm.at[wid, e], ebuf.at[e], esem.at[e])
    for s in range(3):
        pltpu.make_async_copy(sd_hbm.at[wid, s], ebuf.at[s],
                              esem.at[s]).wait()
        pltpu.async_copy(m_hbm.at[ebuf.at[s, 0]], rows.at[s], gsem.at[s])

    @pl.loop(0, NCS, step=NEB)
    def _grp(c0):
        for off in range(NEB):
            cc = c0 + off
            s = off % NSLOT
            f = off % NST
            e = off % NEB
            s3 = (off + 3) % NSLOT
            e3 = (off + 3) % NEB
            e5 = (off + 5) % NEB
            em2 = (off - 2) % NEB

            pltpu.make_async_copy(m_hbm.at[ebuf.at[e, 0]], rows.at[s],
                                  gsem.at[s]).wait()

            @pl.when(cc + 5 < NCS)
            def _pf():
                pltpu.async_copy(sd_hbm.at[wid, cc + 5], ebuf.at[e5],
                                 esem.at[e5])

            @pl.when(cc + 3 < NCS)
            def _gnext():
                pltpu.make_async_copy(sd_hbm.at[wid, cc + 3], ebuf.at[e3],
                                      esem.at[e3]).wait()
                pltpu.async_copy(m_hbm.at[ebuf.at[e3, 0]], rows.at[s3],
                                 gsem.at[s3])

            @pl.when(cc >= 2)
            def _wsc():
                pltpu.make_async_copy(stage.at[f],
                                      agg_sh.at[ebuf.at[em2, 1]],
                                      ssem.at[f]).wait()

            @pl.when(cc == NCS // 2)
            def _rld():
                pltpu.sync_copy(wg_hbm.at[wid, 1], w_v)

            cch = cc % (NCS // 2)

            @plsc.parallel_loop(0, CKS, 16, unroll=2)
            def _mul(k0):
                wv = w_v[cch // 2, pl.ds((cch % 2) * CKS + k0, 16)]
                for kk in range(16):
                    wk = wv[kk]
                    k = k0 + kk
                    for g in range(4):
                        v = rows[s, k, pl.ds(g * 16, 16)]
                        flo = plsc.bitcast(lax.shift_left(v, 16), _f32)
                        fhi = plsc.bitcast(v, _f32)
                        stage[f, k, pl.ds(g * 32, 16)] = flo * wk
                        stage[f, k, pl.ds(g * 32 + 16, 16)] = fhi * wk

            pltpu.async_copy(stage.at[f], agg_sh.at[ebuf.at[e, 1]],
                             ssem.at[f], add=True)

    for cc in range(NCS - 2, NCS):
        f = cc % NST
        e = cc % NEB
        pltpu.make_async_copy(stage.at[f], agg_sh.at[ebuf.at[e, 1]],
                              ssem.at[f]).wait()

    plsc.subcore_barrier()
    for t in range(RPT // CKS):
        pltpu.sync_copy(agg_sh.at[pl.ds(sid * RPT + t * CKS, CKS)],
                        stage.at[0])
        pltpu.sync_copy(stage.at[0],
                        agg_hbm.at[cid, pl.ds(sid * RPT + t * CKS, CKS)])


_sc_scatter = functools.partial(
    pl.kernel,
    out_type=jax.ShapeDtypeStruct((2, NPAD, D), _f32),
    mesh=_mesh,
    compiler_params=pltpu.CompilerParams(use_tc_tiling_on_sc=False,
                                         needs_layout_passes=False),
    scratch_types=[
        pltpu.VMEM_SHARED((NPAD, D), _f32),
        pltpu.VMEM((NCS * CKS // 256, 128), _f32),
        pltpu.VMEM((NSLOT, CKS, D // 2), jnp.int32),
        pltpu.VMEM((NST, CKS, D), _f32),
        pltpu.VMEM((NEB, 2, CKS), jnp.int32),
        pltpu.SemaphoreType.DMA((NSLOT,)),
        pltpu.SemaphoreType.DMA((NST,)),
        pltpu.SemaphoreType.DMA((NEB,)),
    ],
)(_sc_scatter_body)


def _tc_embed_body(z_ref, emb_ref, wm_ref, bm_ref, vdw_ref,
                   h_ref, m_ref, vol_ref):
    z = z_ref[...]
    oh = (z == lax.broadcasted_iota(jnp.int32, (NBLK, NTYPES), 1)
          ).astype(_f32)
    h = jnp.dot(oh, emb_ref[...], preferred_element_type=_f32)
    h_ref[...] = h
    m_ref[...] = jnp.maximum(
        jnp.dot(h, wm_ref[...], preferred_element_type=_f32) + bm_ref[...],
        0.0).astype(jnp.bfloat16)
    r = vdw_ref[...]
    vol_ref[...] = jnp.dot(oh, (4.0 / 3.0) * np.pi * r * r * r,
                           preferred_element_type=_f32)


def _tc_embed(z2, atom_embed, W_msg, b_msg2, vdw2):
    return pl.pallas_call(
        _tc_embed_body,
        grid=(N // NBLK,),
        in_specs=[
            pl.BlockSpec((NBLK, 1), lambda i: (i, 0)),
            pl.BlockSpec((NTYPES, D), lambda i: (0, 0)),
            pl.BlockSpec((D, D), lambda i: (0, 0)),
            pl.BlockSpec((1, D), lambda i: (0, 0)),
            pl.BlockSpec((NTYPES, 1), lambda i: (0, 0)),
        ],
        out_specs=[
            pl.BlockSpec((NBLK, D), lambda i: (i, 0)),
            pl.BlockSpec((NBLK, D), lambda i: (i, 0)),
            pl.BlockSpec((NBLK, 1), lambda i: (i, 0)),
        ],
        out_shape=[
            jax.ShapeDtypeStruct((N, D), _f32),
            jax.ShapeDtypeStruct((N, D), jnp.bfloat16),
            jax.ShapeDtypeStruct((N, 1), _f32),
        ],
    )(z2, atom_embed, W_msg, b_msg2, vdw2)


def _tc_w_body(d2_ref, g_ref, w_ref):
    s = jnp.dot(d2_ref[...], g_ref[...], preferred_element_type=_f32)
    w = jnp.exp(-jnp.sqrt(s))
    i = pl.program_id(0)
    row = lax.broadcasted_iota(jnp.int32, s.shape, 0)
    col = lax.broadcasted_iota(jnp.int32, s.shape, 1)
    e = (i * WBLK + row) * 8 + col
    w_ref[...] = jnp.where(e < E, w, 0.0)


def _tc_w(d2m, gmat):
    return pl.pallas_call(
        _tc_w_body,
        grid=(EPAD // 8 // WBLK,),
        in_specs=[
            pl.BlockSpec((WBLK, 128), lambda i: (i, 0)),
            pl.BlockSpec((128, 8), lambda i: (0, 0)),
        ],
        out_specs=pl.BlockSpec((WBLK, 8), lambda i: (i, 0)),
        out_shape=jax.ShapeDtypeStruct((EPAD // 8, 8), _f32),
    )(d2m, gmat)


def _tc_round_body(agg_ref, h_ref, wu_ref, bu_ref, wm_ref, bm_ref,
                   hn_ref, mn_ref):
    a = agg_ref[0] + agg_ref[1]
    hn = jnp.maximum(
        jnp.dot(a, wu_ref[...], preferred_element_type=_f32)
        + bu_ref[...] + h_ref[...], 0.0)
    hn_ref[...] = hn
    if mn_ref is not None:
        mn_ref[...] = jnp.maximum(
            jnp.dot(hn, wm_ref[...], preferred_element_type=_f32)
            + bm_ref[...], 0.0).astype(jnp.bfloat16)


def _tc_round(agg, h, W_upd, b_upd2, W_msg, b_msg2, last):
    body = (functools.partial(_tc_round_body, mn_ref=None) if last
            else _tc_round_body)
    out_specs = [pl.BlockSpec((NBLK, D), lambda i: (i, 0))]
    out_shape = [jax.ShapeDtypeStruct((N, D), _f32)]
    if not last:
        out_specs.append(pl.BlockSpec((NBLK, D), lambda i: (i, 0)))
        out_shape.append(jax.ShapeDtypeStruct((N, D), jnp.bfloat16))
    return pl.pallas_call(
        body,
        grid=(N // NBLK,),
        in_specs=[
            pl.BlockSpec((2, NBLK, D), lambda i: (0, i, 0)),
            pl.BlockSpec((NBLK, D), lambda i: (i, 0)),
            pl.BlockSpec((D, D), lambda i: (0, 0)),
            pl.BlockSpec((1, D), lambda i: (0, 0)),
            pl.BlockSpec((D, D), lambda i: (0, 0)),
            pl.BlockSpec((1, D), lambda i: (0, 0)),
        ],
        out_specs=out_specs,
        out_shape=out_shape,
    )(agg, h, W_upd, b_upd2, W_msg, b_msg2)


def _tc_pool_body(h_ref, batch_ref, vol_ref, wo_ref, bo_ref,
                  pred_ref, mvol_ref, g_sc, cnt_sc, vol_sc):
    i = pl.program_id(0)

    @pl.when(i == 0)
    def _init():
        g_sc[...] = jnp.zeros_like(g_sc)
        cnt_sc[...] = jnp.zeros_like(cnt_sc)
        vol_sc[...] = jnp.zeros_like(vol_sc)

    ohT = (lax.broadcasted_iota(jnp.int32, (NG, NBLK), 0) == batch_ref[0]
           ).astype(_f32)
    g_sc[...] += jnp.dot(ohT, h_ref[...], preferred_element_type=_f32)
    cnt_sc[...] += jnp.sum(ohT, axis=1, keepdims=True)
    vol_sc[...] += jnp.dot(ohT, vol_ref[...], preferred_element_type=_f32)

    @pl.when(i == N // NBLK - 1)
    def _fin():
        gm = g_sc[...] / jnp.maximum(cnt_sc[...], 1.0)
        pred = jnp.dot(gm, wo_ref[...], preferred_element_type=_f32) \
            + bo_ref[...]
        pred_ref[...] = pred * TSTD + TMEAN
        mvol_ref[...] = vol_sc[...]


def _tc_pool(h, batch2, vol, W_out, b_out2):
    return pl.pallas_call(
        _tc_pool_body,
        grid=(N // NBLK,),
        in_specs=[
            pl.BlockSpec((NBLK, D), lambda i: (i, 0)),
            pl.BlockSpec((1, 1, NBLK), lambda i: (i, 0, 0)),
            pl.BlockSpec((NBLK, 1), lambda i: (i, 0)),
            pl.BlockSpec((D, 1), lambda i: (0, 0)),
            pl.BlockSpec((1, 1), lambda i: (0, 0)),
        ],
        out_specs=[
            pl.BlockSpec((NG, 1), lambda i: (0, 0)),
            pl.BlockSpec((NG, 1), lambda i: (0, 0)),
        ],
        out_shape=[
            jax.ShapeDtypeStruct((NG, 1), _f32),
            jax.ShapeDtypeStruct((NG, 1), _f32),
        ],
        scratch_shapes=[
            pltpu.VMEM((NG, D), _f32),
            pltpu.VMEM((NG, 1), _f32),
            pltpu.VMEM((NG, 1), _f32),
        ],
    )(h, batch2, vol, W_out, b_out2)


_GMAT = np.kron(np.eye(8, dtype=np.float32), np.ones((16, 1), np.float32))

_PERM = np.zeros(128, np.int32)
for _g in range(4):
    for _i in range(16):
        _PERM[32 * _g + 2 * _i] = 32 * _g + _i
        _PERM[32 * _g + 2 * _i + 1] = 32 * _g + 16 + _i


def kernel(z, pos, edge_index, batch, atom_embed, W_msg, b_msg, W_upd, b_upd,
           W_out, b_out, vdw_radii):
    src = edge_index[0].astype(jnp.int32)
    dst = edge_index[1].astype(jnp.int32)
    padn = EPAD - E
    zpad = jnp.zeros((padn,), jnp.int32)
    srcp = jnp.concatenate([src, zpad])
    dstp = jnp.concatenate([dst, zpad])
    srcg = srcp.reshape(NT, NCHUNK, CK)
    dstg = dstp.reshape(NT, NCHUNK, CK)
    pos16 = jnp.pad(pos.astype(_f32), ((0, 0), (0, 13)))
    z2 = z.astype(jnp.int32).reshape(N, 1)
    batch2 = batch.astype(jnp.int32).reshape(N // NBLK, 1, NBLK)
    perm = jnp.asarray(_PERM)
    W_msg = W_msg[:, perm]
    b_msg = b_msg[perm]
    b_msg2 = b_msg.reshape(1, D)
    b_upd2 = b_upd.reshape(1, D)
    b_out2 = b_out.reshape(1, 1)
    vdw2 = vdw_radii.reshape(NTYPES, 1)
    gmat = jnp.asarray(_GMAT)

    d2g = _sc_edge_pos(pos16, srcg, dstg)
    h, m, vol = _tc_embed(z2, atom_embed, W_msg, b_msg2, vdw2)
    wflat = _tc_w(d2g.reshape(EPAD // 8, 128), gmat)
    wgs = wflat.reshape(NT, 2, NCS * CKS // 256, 128)
    sdg = jnp.concatenate([srcp.reshape(NT, NCS, 1, CKS),
                           dstp.reshape(NT, NCS, 1, CKS)], axis=2)

    for r in range(3):
        m_i32 = lax.bitcast_convert_type(
            m.reshape(N, D // 2, 2), jnp.int32)
        agg = _sc_scatter(m_i32, sdg, wgs)
        outs = _tc_round(agg, h, W_upd, b_upd2, W_msg, b_msg2, last=(r == 2))
        if r < 2:
            h, m = outs
        else:
            (h,) = outs

    pred2, mvol2 = _tc_pool(h, batch2, vol, W_out, b_out2)
    return pred2.reshape(NG), mvol2.reshape(NG)


# speedup vs baseline: 1.7132x; 1.1115x over previous
import functools

import numpy as np
import jax
import jax.numpy as jnp
from jax import lax
from jax.experimental import pallas as pl
from jax.experimental.pallas import tpu as pltpu
from jax.experimental.pallas import tpu_sc as plsc

N = 10000
E = 320000
D = 128
NG = 256
NTYPES = 100
TSTD = 0.0271
TMEAN = 0.6226

NT = 32
NCHUNK = 80
CK = 128
NCS = 160
CKS = 64
NSLOT = 4
NST = 2
NEB = 8
EPAD = NT * NCHUNK * CK
NPAD = 10240
RPT = NPAD // 16
ZR = 128

NBLK = 2000
WBLK = 4096

_mesh = plsc.VectorSubcoreMesh(core_axis_name="c", subcore_axis_name="s")
_f32 = jnp.float32


def _sc_edge_pos_body(pos16, srcg, dstg, d2g,
                      src_v, dst_v, av, bv, gsa, gsb, wsem):
    cid = lax.axis_index("c")
    sid = lax.axis_index("s")
    wid = cid * 16 + sid
    pltpu.sync_copy(srcg.at[wid], src_v)
    pltpu.sync_copy(dstg.at[wid], dst_v)

    for s in range(2):
        pltpu.async_copy(pos16.at[src_v.at[s]], av.at[s], gsa.at[s])
        pltpu.async_copy(pos16.at[dst_v.at[s]], bv.at[s], gsb.at[s])

    @pl.loop(0, NCHUNK, step=4)
    def _grp(c0):
        for off in range(4):
            cc = c0 + off
            s = off % 4
            s2 = (off + 2) % 4

            pltpu.make_async_copy(pos16.at[src_v.at[cc]], av.at[s],
                                  gsa.at[s]).wait()
            pltpu.make_async_copy(pos16.at[dst_v.at[cc]], bv.at[s],
                                  gsb.at[s]).wait()

            @pl.when(cc + 2 < NCHUNK)
            def _gnext():
                @pl.when(cc >= 2)
                def _ww():
                    pltpu.make_async_copy(av.at[s2], d2g.at[wid, cc - 2],
                                          wsem.at[s2]).wait()
                pltpu.async_copy(pos16.at[src_v.at[cc + 2]], av.at[s2],
                                 gsa.at[s2])
                pltpu.async_copy(pos16.at[dst_v.at[cc + 2]], bv.at[s2],
                                 gsb.at[s2])

            @plsc.parallel_loop(0, CK, 1, unroll=4)
            def _edge(k):
                dvec = av[s, k, :] - bv[s, k, :]
                av[s, k, :] = dvec * dvec

            pltpu.async_copy(av.at[s], d2g.at[wid, cc], wsem.at[s])

    for cc in range(NCHUNK - 4, NCHUNK):
        s = cc % 4
        pltpu.make_async_copy(av.at[s], d2g.at[wid, cc], wsem.at[s]).wait()


_sc_edge_pos = functools.partial(
    pl.kernel,
    out_type=jax.ShapeDtypeStruct((NT, NCHUNK, CK, 16), _f32),
    mesh=_mesh,
    compiler_params=pltpu.CompilerParams(use_tc_tiling_on_sc=False),
    scratch_types=[
        pltpu.VMEM((NCHUNK, CK), jnp.int32),
        pltpu.VMEM((NCHUNK, CK), jnp.int32),
        pltpu.VMEM((4, CK, 16), _f32),
        pltpu.VMEM((4, CK, 16), _f32),
        pltpu.SemaphoreType.DMA((4,)),
        pltpu.SemaphoreType.DMA((4,)),
        pltpu.SemaphoreType.DMA((4,)),
    ],
)(_sc_edge_pos_body)


def _sc_scatter_body(m_hbm, sd_hbm, wg_hbm, agg_hbm,
                     agg_sh, w_v, rows, stage, ebuf, gsem, ssem, esem):
    cid = lax.axis_index("c")
    sid = lax.axis_index("s")
    wid = cid * 16 + sid

    @pl.loop(0, CKS)
    def _zrow(r):
        for j in range(8):
            stage[0, r, pl.ds(j * 16, 16)] = jnp.zeros((16,), _f32)

    for t in range(RPT // CKS):
        pltpu.sync_copy(stage.at[0],
                        agg_sh.at[pl.ds(sid * RPT + t * CKS, CKS)])
    plsc.subcore_barrier()

    pltpu.sync_copy(wg_hbm.at[wid, 0], w_v)
    for e in range(5):
        pltpu.async_copy(sd_hbm.at[wid, e], ebuf.at[e], esem.at[e])
    for s in range(3):
        pltpu.make_async_copy(sd_hbm.at[wid, s], ebuf.at[s],
                              esem.at[s]).wait()
        pltpu.async_copy(m_hbm.at[ebuf.at[s, 0]], rows.at[s], gsem.at[s])

    @pl.loop(0, NCS, step=NEB)
    def _grp(c0):
        for off in range(NEB):
            cc = c0 + off
            s = off % NSLOT
            f = off % NST
            e = off % NEB
            s3 = (off + 3) % NSLOT
            e3 = (off + 3) % NEB
            e5 = (off + 5) % NEB
            em2 = (off - 2) % NEB

            pltpu.make_async_copy(m_hbm.at[ebuf.at[e, 0]], rows.at[s],
                                  gsem.at[s]).wait()

            @pl.when(cc + 5 < NCS)
            def _pf():
                pltpu.async_copy(sd_hbm.at[wid, cc + 5], ebuf.at[e5],
                                 esem.at[e5])

            @pl.when(cc + 3 < NCS)
            def _gnext():
                pltpu.make_async_copy(sd_hbm.at[wid, cc + 3], ebuf.at[e3],
                                      esem.at[e3]).wait()
                pltpu.async_copy(m_hbm.at[ebuf.at[e3, 0]], rows.at[s3],
                                 gsem.at[s3])

            @pl.when(cc >= 2)
            def _wsc():
                pltpu.make_async_copy(stage.at[f],
                                      agg_sh.at[ebuf.at[em2, 1]],
                                      ssem.at[f]).wait()

            @pl.when(cc == NCS // 2)
            def _rld():
                pltpu.sync_copy(wg_hbm.at[wid, 1], w_v)

            cch = cc % (NCS // 2)

            @plsc.parallel_loop(0, CKS, 16, unroll=2)
            def _mul(k0):
                wv = w_v[cch // 2, pl.ds((cch % 2) * CKS + k0, 16)]
                for kk in range(16):
                    wk = wv[kk]
                    k = k0 + kk
                    for g in range(4):
                        v = rows[s, k, pl.ds(g * 16, 16)]
                        flo = plsc.bitcast(lax.shift_left(v, 16), _f32)
                        fhi = plsc.bitcast(v, _f32)
                        stage[f, k, pl.ds(g * 32, 16)] = flo * wk
                        stage[f, k, pl.ds(g * 32 + 16, 16)] = fhi * wk

            pltpu.async_copy(stage.at[f], agg_sh.at[ebuf.at[e, 1]],
                             ssem.at[f], add=True)

    for cc in range(NCS - 2, NCS):
        f = cc % NST
        e = cc % NEB
        pltpu.make_async_copy(stage.at[f], agg_sh.at[ebuf.at[e, 1]],
                              ssem.at[f]).wait()

    plsc.subcore_barrier()
    for t in range(RPT // CKS):
        pltpu.sync_copy(agg_sh.at[pl.ds(sid * RPT + t * CKS, CKS)],
                        stage.at[0])
        pltpu.sync_copy(stage.at[0],
                        agg_hbm.at[cid, pl.ds(sid * RPT + t * CKS, CKS)])


_sc_scatter = functools.partial(
    pl.kernel,
    out_type=jax.ShapeDtypeStruct((2, NPAD, D), _f32),
    mesh=_mesh,
    compiler_params=pltpu.CompilerParams(use_tc_tiling_on_sc=False,
                                         needs_layout_passes=False),
    scratch_types=[
        pltpu.VMEM_SHARED((NPAD, D), _f32),
        pltpu.VMEM((NCS * CKS // 256, 128), _f32),
        pltpu.VMEM((NSLOT, CKS, D // 2), jnp.int32),
        pltpu.VMEM((NST, CKS, D), _f32),
        pltpu.VMEM((NEB, 2, CKS), jnp.int32),
        pltpu.SemaphoreType.DMA((NSLOT,)),
        pltpu.SemaphoreType.DMA((NST,)),
        pltpu.SemaphoreType.DMA((NEB,)),
    ],
)(_sc_scatter_body)


def _tc_embed_body(z_ref, emb_ref, wm_ref, bm_ref, vdw_ref,
                   h_ref, m_ref, vol_ref):
    z = z_ref[...]
    oh = (z == lax.broadcasted_iota(jnp.int32, (NBLK, NTYPES), 1)
          ).astype(_f32)
    h = jnp.dot(oh, emb_ref[...], preferred_element_type=_f32)
    h_ref[...] = h
    m_ref[...] = jnp.maximum(
        jnp.dot(h, wm_ref[...], preferred_element_type=_f32) + bm_ref[...],
        0.0).astype(jnp.bfloat16)
    r = vdw_ref[...]
    vol_ref[...] = jnp.dot(oh, (4.0 / 3.0) * np.pi * r * r * r,
                           preferred_element_type=_f32)


def _tc_embed(z2, atom_embed, W_msg, b_msg2, vdw2):
    return pl.pallas_call(
        _tc_embed_body,
        grid=(N // NBLK,),
        in_specs=[
            pl.BlockSpec((NBLK, 1), lambda i: (i, 0)),
            pl.BlockSpec((NTYPES, D), lambda i: (0, 0)),
            pl.BlockSpec((D, D), lambda i: (0, 0)),
            pl.BlockSpec((1, D), lambda i: (0, 0)),
            pl.BlockSpec((NTYPES, 1), lambda i: (0, 0)),
        ],
        out_specs=[
            pl.BlockSpec((NBLK, D), lambda i: (i, 0)),
            pl.BlockSpec((NBLK, D), lambda i: (i, 0)),
            pl.BlockSpec((NBLK, 1), lambda i: (i, 0)),
        ],
        out_shape=[
            jax.ShapeDtypeStruct((N, D), _f32),
            jax.ShapeDtypeStruct((N, D), jnp.bfloat16),
            jax.ShapeDtypeStruct((N, 1), _f32),
        ],
    )(z2, atom_embed, W_msg, b_msg2, vdw2)


def _tc_w_body(d2_ref, g_ref, w_ref):
    s = jnp.dot(d2_ref[...], g_ref[...], preferred_element_type=_f32)
    w = jnp.exp(-jnp.sqrt(s))
    i = pl.program_id(0)
    row = lax.broadcasted_iota(jnp.int32, s.shape, 0)
    col = lax.broadcasted_iota(jnp.int32, s.shape, 1)
    e = (i * WBLK + row) * 8 + col
    w_ref[...] = jnp.where(e < E, w, 0.0)


def _tc_w(d2m, gmat):
    return pl.pallas_call(
        _tc_w_body,
        grid=(EPAD // 8 // WBLK,),
        in_specs=[
            pl.BlockSpec((WBLK, 128), lambda i: (i, 0)),
            pl.BlockSpec((128, 8), lambda i: (0, 0)),
        ],
        out_specs=pl.BlockSpec((WBLK, 8), lambda i: (i, 0)),
        out_shape=jax.ShapeDtypeStruct((EPAD // 8, 8), _f32),
    )(d2m, gmat)


def _tc_round_body(agg_ref, h_ref, wu_ref, bu_ref, wm_ref, bm_ref,
                   hn_ref, mn_ref):
    a = agg_ref[0] + agg_ref[1]
    hn = jnp.maximum(
        jnp.dot(a, wu_ref[...], preferred_element_type=_f32)
        + bu_ref[...] + h_ref[...], 0.0)
    hn_ref[...] = hn
    if mn_ref is not None:
        mn_ref[...] = jnp.maximum(
            jnp.dot(hn, wm_ref[...], preferred_element_type=_f32)
            + bm_ref[...], 0.0).astype(jnp.bfloat16)


def _tc_round(agg, h, W_upd, b_upd2, W_msg, b_msg2, last):
    body = (functools.partial(_tc_round_body, mn_ref=None) if last
            else _tc_round_body)
    out_specs = [pl.BlockSpec((NBLK, D), lambda i: (i, 0))]
    out_shape = [jax.ShapeDtypeStruct((N, D), _f32)]
    if not last:
        out_specs.append(pl.BlockSpec((NBLK, D), lambda i: (i, 0)))
        out_shape.append(jax.ShapeDtypeStruct((N, D), jnp.bfloat16))
    return pl.pallas_call(
        body,
        grid=(N // NBLK,),
        in_specs=[
            pl.BlockSpec((2, NBLK, D), lambda i: (0, i, 0)),
            pl.BlockSpec((NBLK, D), lambda i: (i, 0)),
            pl.BlockSpec((D, D), lambda i: (0, 0)),
            pl.BlockSpec((1, D), lambda i: (0, 0)),
            pl.BlockSpec((D, D), lambda i: (0, 0)),
            pl.BlockSpec((1, D), lambda i: (0, 0)),
        ],
        out_specs=out_specs,
        out_shape=out_shape,
    )(agg, h, W_upd, b_upd2, W_msg, b_msg2)


def _tc_pool_body(h_ref, batch_ref, vol_ref, wo_ref, bo_ref,
                  pred_ref, mvol_ref, g_sc, cnt_sc, vol_sc):
    i = pl.program_id(0)

    @pl.when(i == 0)
    def _init():
        g_sc[...] = jnp.zeros_like(g_sc)
        cnt_sc[...] = jnp.zeros_like(cnt_sc)
        vol_sc[...] = jnp.zeros_like(vol_sc)

    ohT = (lax.broadcasted_iota(jnp.int32, (NG, NBLK), 0) == batch_ref[0]
           ).astype(_f32)
    g_sc[...] += jnp.dot(ohT, h_ref[...], preferred_element_type=_f32)
    cnt_sc[...] += jnp.sum(ohT, axis=1, keepdims=True)
    vol_sc[...] += jnp.dot(ohT, vol_ref[...], preferred_element_type=_f32)

    @pl.when(i == N // NBLK - 1)
    def _fin():
        gm = g_sc[...] / jnp.maximum(cnt_sc[...], 1.0)
        pred = jnp.dot(gm, wo_ref[...], preferred_element_type=_f32) \
            + bo_ref[...]
        pred_ref[...] = pred * TSTD + TMEAN
        mvol_ref[...] = vol_sc[...]


def _tc_pool(h, batch2, vol, W_out, b_out2):
    return pl.pallas_call(
        _tc_pool_body,
        grid=(N // NBLK,),
        in_specs=[
            pl.BlockSpec((NBLK, D), lambda i: (i, 0)),
            pl.BlockSpec((1, 1, NBLK), lambda i: (i, 0, 0)),
            pl.BlockSpec((NBLK, 1), lambda i: (i, 0)),
            pl.BlockSpec((D, 1), lambda i: (0, 0)),
            pl.BlockSpec((1, 1), lambda i: (0, 0)),
        ],
        out_specs=[
            pl.BlockSpec((NG, 1), lambda i: (0, 0)),
            pl.BlockSpec((NG, 1), lambda i: (0, 0)),
        ],
        out_shape=[
            jax.ShapeDtypeStruct((NG, 1), _f32),
            jax.ShapeDtypeStruct((NG, 1), _f32),
        ],
        scratch_shapes=[
            pltpu.VMEM((NG, D), _f32),
            pltpu.VMEM((NG, 1), _f32),
            pltpu.VMEM((NG, 1), _f32),
        ],
    )(h, batch2, vol, W_out, b_out2)


_GMAT = np.kron(np.eye(8, dtype=np.float32), np.ones((16, 1), np.float32))

_PERM = np.zeros(128, np.int32)
for _g in range(4):
    for _i in range(16):
        _PERM[32 * _g + 2 * _i] = 32 * _g + _i
        _PERM[32 * _g + 2 * _i + 1] = 32 * _g + 16 + _i


def kernel(z, pos, edge_index, batch, atom_embed, W_msg, b_msg, W_upd, b_upd,
           W_out, b_out, vdw_radii):
    src = edge_index[0].astype(jnp.int32)
    dst = edge_index[1].astype(jnp.int32)
    padn = EPAD - E
    zpad = jnp.zeros((padn,), jnp.int32)
    srcp = jnp.concatenate([src, zpad])
    dstp = jnp.concatenate([dst, zpad])
    srcg = srcp.reshape(NT, NCHUNK, CK)
    dstg = dstp.reshape(NT, NCHUNK, CK)
    pos16 = jnp.pad(pos.astype(_f32), ((0, 0), (0, 13)))
    z2 = z.astype(jnp.int32).reshape(N, 1)
    batch2 = batch.astype(jnp.int32).reshape(N // NBLK, 1, NBLK)
    perm = jnp.asarray(_PERM)
    W_msg = W_msg[:, perm]
    b_msg = b_msg[perm]
    b_msg2 = b_msg.reshape(1, D)
    b_upd2 = b_upd.reshape(1, D)
    b_out2 = b_out.reshape(1, 1)
    vdw2 = vdw_radii.reshape(NTYPES, 1)
    gmat = jnp.asarray(_GMAT)

    d2g = _sc_edge_pos(pos16, srcg, dstg)
    h, m, vol = _tc_embed(z2, atom_embed, W_msg, b_msg2, vdw2)
    wflat = _tc_w(d2g.reshape(EPAD // 8, 128), gmat)
    wgs = wflat.reshape(NT, 2, NCS * CKS // 256, 128)
    sdg = jnp.concatenate([srcp.reshape(NT, NCS, 1, CKS),
                           dstp.reshape(NT, NCS, 1, CKS)], axis=2)

    for r in range(3):
        m_i32 = lax.bitcast_convert_type(
            m.reshape(N, D // 2, 2), jnp.int32)
        agg = _sc_scatter(m_i32, sdg, wgs)
        outs = _tc_round(agg, h, W_upd, b_upd2, W_msg, b_msg2, last=(r == 2))
        if r < 2:
            h, m = outs
        else:
            (h,) = outs

    pred2, mvol2 = _tc_pool(h, batch2, vol, W_out, b_out2)
    return pred2.reshape(NG), mvol2.reshape(NG)


# speedup vs baseline: 1.7198x; 1.0039x over previous
import functools

import numpy as np
import jax
import jax.numpy as jnp
from jax import lax
from jax.experimental import pallas as pl
from jax.experimental.pallas import tpu as pltpu
from jax.experimental.pallas import tpu_sc as plsc

N = 10000
E = 320000
D = 128
NG = 256
NTYPES = 100
TSTD = 0.0271
TMEAN = 0.6226

NT = 32
NCHUNK = 80
CK = 128
NCS = 160
CKS = 64
NSLOT = 4
NST = 2
NEB = 8
EPAD = NT * NCHUNK * CK
NPAD = 10240
RPT = NPAD // 16
ZR = 128

NBLK = 2000
WBLK = 4096

_mesh = plsc.VectorSubcoreMesh(core_axis_name="c", subcore_axis_name="s")
_f32 = jnp.float32


def _sc_edge_pos_body(pos16, srcg, dstg, d2g,
                      src_v, dst_v, av, bv, gsa, gsb, wsem):
    cid = lax.axis_index("c")
    sid = lax.axis_index("s")
    wid = cid * 16 + sid
    pltpu.sync_copy(srcg.at[wid], src_v)
    pltpu.sync_copy(dstg.at[wid], dst_v)

    for s in range(6):
        pltpu.async_copy(pos16.at[src_v.at[s]], av.at[s], gsa.at[s])
        pltpu.async_copy(pos16.at[dst_v.at[s]], bv.at[s], gsb.at[s])

    @pl.loop(0, NCHUNK, step=8)
    def _grp(c0):
        for off in range(8):
            cc = c0 + off
            s = off % 8
            s2 = (off + 6) % 8

            pltpu.make_async_copy(pos16.at[src_v.at[cc]], av.at[s],
                                  gsa.at[s]).wait()
            pltpu.make_async_copy(pos16.at[dst_v.at[cc]], bv.at[s],
                                  gsb.at[s]).wait()

            @pl.when(cc + 6 < NCHUNK)
            def _gnext():
                @pl.when(cc >= 2)
                def _ww():
                    pltpu.make_async_copy(av.at[s2], d2g.at[wid, cc - 2],
                                          wsem.at[s2]).wait()
                pltpu.async_copy(pos16.at[src_v.at[cc + 6]], av.at[s2],
                                 gsa.at[s2])
                pltpu.async_copy(pos16.at[dst_v.at[cc + 6]], bv.at[s2],
                                 gsb.at[s2])

            @plsc.parallel_loop(0, CK, 1, unroll=4)
            def _edge(k):
                dvec = av[s, k, :] - bv[s, k, :]
                av[s, k, :] = dvec * dvec

            pltpu.async_copy(av.at[s], d2g.at[wid, cc], wsem.at[s])

    for cc in range(NCHUNK - 8, NCHUNK):
        s = cc % 8
        pltpu.make_async_copy(av.at[s], d2g.at[wid, cc], wsem.at[s]).wait()


_sc_edge_pos = functools.partial(
    pl.kernel,
    out_type=jax.ShapeDtypeStruct((NT, NCHUNK, CK, 16), _f32),
    mesh=_mesh,
    compiler_params=pltpu.CompilerParams(use_tc_tiling_on_sc=False),
    scratch_types=[
        pltpu.VMEM((NCHUNK, CK), jnp.int32),
        pltpu.VMEM((NCHUNK, CK), jnp.int32),
        pltpu.VMEM((8, CK, 16), _f32),
        pltpu.VMEM((8, CK, 16), _f32),
        pltpu.SemaphoreType.DMA((8,)),
        pltpu.SemaphoreType.DMA((8,)),
        pltpu.SemaphoreType.DMA((8,)),
    ],
)(_sc_edge_pos_body)


def _sc_scatter_body(m_hbm, sd_hbm, wg_hbm, agg_hbm,
                     agg_sh, w_v, rows, stage, ebuf, gsem, ssem, esem):
    cid = lax.axis_index("c")
    sid = lax.axis_index("s")
    wid = cid * 16 + sid

    @pl.loop(0, CKS)
    def _zrow(r):
        for j in range(8):
            stage[0, r, pl.ds(j * 16, 16)] = jnp.zeros((16,), _f32)

    for t in range(RPT // CKS):
        pltpu.sync_copy(stage.at[0],
                        agg_sh.at[pl.ds(sid * RPT + t * CKS, CKS)])
    plsc.subcore_barrier()

    pltpu.sync_copy(wg_hbm.at[wid, 0], w_v)
    for e in range(5):
        pltpu.async_copy(sd_hbm.at[wid, e], ebuf.at[e], esem.at[e])
    for s in range(3):
        pltpu.make_async_copy(sd_hbm.at[wid, s], ebuf.at[s],
                              esem.at[s]).wait()
        pltpu.async_copy(m_hbm.at[ebuf.at[s, 0]], rows.at[s], gsem.at[s])

    @pl.loop(0, NCS, step=NEB)
    def _grp(c0):
        for off in range(NEB):
            cc = c0 + off
            s = off % NSLOT
            f = off % NST
            e = off % NEB
            s3 = (off + 3) % NSLOT
            e3 = (off + 3) % NEB
            e5 = (off + 5) % NEB
            em2 = (off - 2) % NEB

            pltpu.make_async_copy(m_hbm.at[ebuf.at[e, 0]], rows.at[s],
                                  gsem.at[s]).wait()

            @pl.when(cc + 5 < NCS)
            def _pf():
                pltpu.async_copy(sd_hbm.at[wid, cc + 5], ebuf.at[e5],
                                 esem.at[e5])

            @pl.when(cc + 3 < NCS)
            def _gnext():
                pltpu.make_async_copy(sd_hbm.at[wid, cc + 3], ebuf.at[e3],
                                      esem.at[e3]).wait()
                pltpu.async_copy(m_hbm.at[ebuf.at[e3, 0]], rows.at[s3],
                                 gsem.at[s3])

            @pl.when(cc >= 2)
            def _wsc():
                pltpu.make_async_copy(stage.at[f],
                                      agg_sh.at[ebuf.at[em2, 1]],
                                      ssem.at[f]).wait()

            @pl.when(cc == NCS // 2)
            def _rld():
                pltpu.sync_copy(wg_hbm.at[wid, 1], w_v)

            cch = cc % (NCS // 2)

            @plsc.parallel_loop(0, CKS, 16, unroll=2)
            def _mul(k0):
                wv = w_v[cch // 2, pl.ds((cch % 2) * CKS + k0, 16)]
                for kk in range(16):
                    wk = wv[kk]
                    k = k0 + kk
                    for g in range(4):
                        v = rows[s, k, pl.ds(g * 16, 16)]
                        flo = plsc.bitcast(lax.shift_left(v, 16), _f32)
                        fhi = plsc.bitcast(v, _f32)
                        stage[f, k, pl.ds(g * 32, 16)] = flo * wk
                        stage[f, k, pl.ds(g * 32 + 16, 16)] = fhi * wk

            pltpu.async_copy(stage.at[f], agg_sh.at[ebuf.at[e, 1]],
                             ssem.at[f], add=True)

    for cc in range(NCS - 2, NCS):
        f = cc % NST
        e = cc % NEB
        pltpu.make_async_copy(stage.at[f], agg_sh.at[ebuf.at[e, 1]],
                              ssem.at[f]).wait()

    plsc.subcore_barrier()
    for t in range(RPT // CKS):
        pltpu.sync_copy(agg_sh.at[pl.ds(sid * RPT + t * CKS, CKS)],
                        stage.at[0])
        pltpu.sync_copy(stage.at[0],
                        agg_hbm.at[cid, pl.ds(sid * RPT + t * CKS, CKS)])


_sc_scatter = functools.partial(
    pl.kernel,
    out_type=jax.ShapeDtypeStruct((2, NPAD, D), _f32),
    mesh=_mesh,
    compiler_params=pltpu.CompilerParams(use_tc_tiling_on_sc=False,
                                         needs_layout_passes=False),
    scratch_types=[
        pltpu.VMEM_SHARED((NPAD, D), _f32),
        pltpu.VMEM((NCS * CKS // 256, 128), _f32),
        pltpu.VMEM((NSLOT, CKS, D // 2), jnp.int32),
        pltpu.VMEM((NST, CKS, D), _f32),
        pltpu.VMEM((NEB, 2, CKS), jnp.int32),
        pltpu.SemaphoreType.DMA((NSLOT,)),
        pltpu.SemaphoreType.DMA((NST,)),
        pltpu.SemaphoreType.DMA((NEB,)),
    ],
)(_sc_scatter_body)


def _tc_embed_body(z_ref, emb_ref, wm_ref, bm_ref, vdw_ref,
                   h_ref, m_ref, vol_ref):
    z = z_ref[...]
    oh = (z == lax.broadcasted_iota(jnp.int32, (NBLK, NTYPES), 1)
          ).astype(_f32)
    h = jnp.dot(oh, emb_ref[...], preferred_element_type=_f32)
    h_ref[...] = h
    m_ref[...] = jnp.maximum(
        jnp.dot(h, wm_ref[...], preferred_element_type=_f32) + bm_ref[...],
        0.0).astype(jnp.bfloat16)
    r = vdw_ref[...]
    vol_ref[...] = jnp.dot(oh, (4.0 / 3.0) * np.pi * r * r * r,
                           preferred_element_type=_f32)


def _tc_embed(z2, atom_embed, W_msg, b_msg2, vdw2):
    return pl.pallas_call(
        _tc_embed_body,
        grid=(N // NBLK,),
        in_specs=[
            pl.BlockSpec((NBLK, 1), lambda i: (i, 0)),
            pl.BlockSpec((NTYPES, D), lambda i: (0, 0)),
            pl.BlockSpec((D, D), lambda i: (0, 0)),
            pl.BlockSpec((1, D), lambda i: (0, 0)),
            pl.BlockSpec((NTYPES, 1), lambda i: (0, 0)),
        ],
        out_specs=[
            pl.BlockSpec((NBLK, D), lambda i: (i, 0)),
            pl.BlockSpec((NBLK, D), lambda i: (i, 0)),
            pl.BlockSpec((NBLK, 1), lambda i: (i, 0)),
        ],
        out_shape=[
            jax.ShapeDtypeStruct((N, D), _f32),
            jax.ShapeDtypeStruct((N, D), jnp.bfloat16),
            jax.ShapeDtypeStruct((N, 1), _f32),
        ],
    )(z2, atom_embed, W_msg, b_msg2, vdw2)


def _tc_w_body(d2_ref, g_ref, w_ref):
    s = jnp.dot(d2_ref[...], g_ref[...], preferred_element_type=_f32)
    w = jnp.exp(-jnp.sqrt(s))
    i = pl.program_id(0)
    row = lax.broadcasted_iota(jnp.int32, s.shape, 0)
    col = lax.broadcasted_iota(jnp.int32, s.shape, 1)
    e = (i * WBLK + row) * 8 + col
    w_ref[...] = jnp.where(e < E, w, 0.0)


def _tc_w(d2m, gmat):
    return pl.pallas_call(
        _tc_w_body,
        grid=(EPAD // 8 // WBLK,),
        in_specs=[
            pl.BlockSpec((WBLK, 128), lambda i: (i, 0)),
            pl.BlockSpec((128, 8), lambda i: (0, 0)),
        ],
        out_specs=pl.BlockSpec((WBLK, 8), lambda i: (i, 0)),
        out_shape=jax.ShapeDtypeStruct((EPAD // 8, 8), _f32),
    )(d2m, gmat)


def _tc_round_body(agg_ref, h_ref, wu_ref, bu_ref, wm_ref, bm_ref,
                   hn_ref, mn_ref):
    a = agg_ref[0] + agg_ref[1]
    hn = jnp.maximum(
        jnp.dot(a, wu_ref[...], preferred_element_type=_f32)
        + bu_ref[...] + h_ref[...], 0.0)
    hn_ref[...] = hn
    if mn_ref is not None:
        mn_ref[...] = jnp.maximum(
            jnp.dot(hn, wm_ref[...], preferred_element_type=_f32)
            + bm_ref[...], 0.0).astype(jnp.bfloat16)


def _tc_round(agg, h, W_upd, b_upd2, W_msg, b_msg2, last):
    body = (functools.partial(_tc_round_body, mn_ref=None) if last
            else _tc_round_body)
    out_specs = [pl.BlockSpec((NBLK, D), lambda i: (i, 0))]
    out_shape = [jax.ShapeDtypeStruct((N, D), _f32)]
    if not last:
        out_specs.append(pl.BlockSpec((NBLK, D), lambda i: (i, 0)))
        out_shape.append(jax.ShapeDtypeStruct((N, D), jnp.bfloat16))
    return pl.pallas_call(
        body,
        grid=(N // NBLK,),
        in_specs=[
            pl.BlockSpec((2, NBLK, D), lambda i: (0, i, 0)),
            pl.BlockSpec((NBLK, D), lambda i: (i, 0)),
            pl.BlockSpec((D, D), lambda i: (0, 0)),
            pl.BlockSpec((1, D), lambda i: (0, 0)),
            pl.BlockSpec((D, D), lambda i: (0, 0)),
            pl.BlockSpec((1, D), lambda i: (0, 0)),
        ],
        out_specs=out_specs,
        out_shape=out_shape,
    )(agg, h, W_upd, b_upd2, W_msg, b_msg2)


def _tc_pool_body(h_ref, batch_ref, vol_ref, wo_ref, bo_ref,
                  pred_ref, mvol_ref, g_sc, cnt_sc, vol_sc):
    i = pl.program_id(0)

    @pl.when(i == 0)
    def _init():
        g_sc[...] = jnp.zeros_like(g_sc)
        cnt_sc[...] = jnp.zeros_like(cnt_sc)
        vol_sc[...] = jnp.zeros_like(vol_sc)

    ohT = (lax.broadcasted_iota(jnp.int32, (NG, NBLK), 0) == batch_ref[0]
           ).astype(_f32)
    g_sc[...] += jnp.dot(ohT, h_ref[...], preferred_element_type=_f32)
    cnt_sc[...] += jnp.sum(ohT, axis=1, keepdims=True)
    vol_sc[...] += jnp.dot(ohT, vol_ref[...], preferred_element_type=_f32)

    @pl.when(i == N // NBLK - 1)
    def _fin():
        gm = g_sc[...] / jnp.maximum(cnt_sc[...], 1.0)
        pred = jnp.dot(gm, wo_ref[...], preferred_element_type=_f32) \
            + bo_ref[...]
        pred_ref[...] = pred * TSTD + TMEAN
        mvol_ref[...] = vol_sc[...]


def _tc_pool(h, batch2, vol, W_out, b_out2):
    return pl.pallas_call(
        _tc_pool_body,
        grid=(N // NBLK,),
        in_specs=[
            pl.BlockSpec((NBLK, D), lambda i: (i, 0)),
            pl.BlockSpec((1, 1, NBLK), lambda i: (i, 0, 0)),
            pl.BlockSpec((NBLK, 1), lambda i: (i, 0)),
            pl.BlockSpec((D, 1), lambda i: (0, 0)),
            pl.BlockSpec((1, 1), lambda i: (0, 0)),
        ],
        out_specs=[
            pl.BlockSpec((NG, 1), lambda i: (0, 0)),
            pl.BlockSpec((NG, 1), lambda i: (0, 0)),
        ],
        out_shape=[
            jax.ShapeDtypeStruct((NG, 1), _f32),
            jax.ShapeDtypeStruct((NG, 1), _f32),
        ],
        scratch_shapes=[
            pltpu.VMEM((NG, D), _f32),
            pltpu.VMEM((NG, 1), _f32),
            pltpu.VMEM((NG, 1), _f32),
        ],
    )(h, batch2, vol, W_out, b_out2)


_GMAT = np.kron(np.eye(8, dtype=np.float32), np.ones((16, 1), np.float32))

_PERM = np.zeros(128, np.int32)
for _g in range(4):
    for _i in range(16):
        _PERM[32 * _g + 2 * _i] = 32 * _g + _i
        _PERM[32 * _g + 2 * _i + 1] = 32 * _g + 16 + _i


def kernel(z, pos, edge_index, batch, atom_embed, W_msg, b_msg, W_upd, b_upd,
           W_out, b_out, vdw_radii):
    src = edge_index[0].astype(jnp.int32)
    dst = edge_index[1].astype(jnp.int32)
    padn = EPAD - E
    zpad = jnp.zeros((padn,), jnp.int32)
    srcp = jnp.concatenate([src, zpad])
    dstp = jnp.concatenate([dst, zpad])
    srcg = srcp.reshape(NT, NCHUNK, CK)
    dstg = dstp.reshape(NT, NCHUNK, CK)
    pos16 = jnp.pad(pos.astype(_f32), ((0, 0), (0, 13)))
    z2 = z.astype(jnp.int32).reshape(N, 1)
    batch2 = batch.astype(jnp.int32).reshape(N // NBLK, 1, NBLK)
    perm = jnp.asarray(_PERM)
    W_msg = W_msg[:, perm]
    b_msg = b_msg[perm]
    b_msg2 = b_msg.reshape(1, D)
    b_upd2 = b_upd.reshape(1, D)
    b_out2 = b_out.reshape(1, 1)
    vdw2 = vdw_radii.reshape(NTYPES, 1)
    gmat = jnp.asarray(_GMAT)

    d2g = _sc_edge_pos(pos16, srcg, dstg)
    h, m, vol = _tc_embed(z2, atom_embed, W_msg, b_msg2, vdw2)
    wflat = _tc_w(d2g.reshape(EPAD // 8, 128), gmat)
    wgs = wflat.reshape(NT, 2, NCS * CKS // 256, 128)
    sdg = jnp.concatenate([srcp.reshape(NT, NCS, 1, CKS),
                           dstp.reshape(NT, NCS, 1, CKS)], axis=2)

    for r in range(3):
        m_i32 = lax.bitcast_convert_type(
            m.reshape(N, D // 2, 2), jnp.int32)
        agg = _sc_scatter(m_i32, sdg, wgs)
        outs = _tc_round(agg, h, W_upd, b_upd2, W_msg, b_msg2, last=(r == 2))
        if r < 2:
            h, m = outs
        else:
            (h,) = outs

    pred2, mvol2 = _tc_pool(h, batch2, vol, W_out, b_out2)
    return pred2.reshape(NG), mvol2.reshape(NG)
